# Initial kernel scaffold; baseline (speedup 1.0000x reference)
import jax
import jax.numpy as jnp
from jax.experimental import pallas as pl


def kernel(x, edge_index, batch, conv_W1, conv_b1, att_W1, att_b1, score_W1, score_b1, conv_W2, conv_b2, att_W2, att_b2, score_W2, score_b2, conv_W3, conv_b3, att_W3, att_b3, score_W3, score_b3, lin1_W, lin1_b, lin2_W, lin2_b, lin3_W, lin3_b):
    raise NotImplementedError("write your pallas kernel here")


# speedup vs baseline: 98.0726x; 98.0726x over previous
import functools
import math

import jax
import jax.numpy as jnp
from jax import lax
from jax.experimental import pallas as pl
from jax.experimental.pallas import tpu as pltpu
from jax.experimental.pallas import tpu_sc as plsc

N0 = 10000
E0 = 320000
D = 128
G = 8
NW = 32
EP = 327680
EW = EP // NW
NBLK = EW // 128
TRASH = 128

STAGES = (
    dict(Np=10240, stride=1250, n_real=1250, k=625, kp_next=640, L=1280),
    dict(Np=5120, stride=640, n_real=625, k=313, kp_next=320, L=640),
    dict(Np=2560, stride=320, n_real=313, k=157, kp_next=160, L=320),
)

_MESH = dict(
    mesh=plsc.VectorSubcoreMesh(core_axis_name="c", subcore_axis_name="s"),
    compiler_params=pltpu.CompilerParams(
        use_tc_tiling_on_sc=False, needs_layout_passes=False),
)


def _worker_ids():
    c = lax.axis_index("c")
    s = lax.axis_index("s")
    return c, s, s * 2 + c


def _zero_rows(zbuf, shared, row0, nrows):
    nfull, rem = nrows // 16, nrows % 16

    def zrow(r, carry):
        pltpu.sync_copy(zbuf, shared.at[pl.ds(row0 + r * 16, 16), :])
        return carry

    lax.fori_loop(0, nfull, zrow, 0)
    if rem:
        pltpu.sync_copy(zbuf.at[pl.ds(0, rem), :],
                        shared.at[pl.ds(row0 + nfull * 16, rem), :])


def _zero_flat(zbuf, shared, off, n):
    nfull, rem = n // 128, n % 128
    for r in range(nfull):
        pltpu.sync_copy(zbuf.at[0, :], shared.at[pl.ds(off + r * 128, 128)])
    if rem:
        pltpu.sync_copy(zbuf.at[0, pl.ds(0, rem)],
                        shared.at[pl.ds(off + nfull * 128, rem)])


def _fill_zbuf(zbuf):
    zeros = jnp.zeros((16,), jnp.float32)

    def body(r, carry):
        for c in range(8):
            zbuf[r, pl.ds(c * 16, 16)] = zeros
        return carry

    lax.fori_loop(0, 16, body, 0)


def _make_pass_a(Np, remap, ni_len):
    acc_rows = Np + TRASH
    epw = acc_rows // 16

    out_type = [jax.ShapeDtypeStruct((2 * acc_rows,), jnp.float32)]
    if remap:
        out_type = [jax.ShapeDtypeStruct((EP,), jnp.int32),
                    jax.ShapeDtypeStruct((EP,), jnp.int32)] + out_type

    scratch = [
        pltpu.VMEM((EW,), jnp.int32),
        pltpu.VMEM((EW,), jnp.int32),
        pltpu.VMEM((1, 128), jnp.int32),
        pltpu.VMEM((1, 128), jnp.float32),
        pltpu.VMEM((16, 128), jnp.float32),
        pltpu.VMEM((epw,), jnp.float32),
    ]
    if remap:
        scratch = [pltpu.VMEM((ni_len,), jnp.int32)] + scratch

    shared = [pltpu.VMEM_SHARED((acc_rows,), jnp.float32)]

    @functools.partial(pl.kernel, out_type=out_type,
                       scratch_types=scratch + shared, **_MESH)
    def pass_a(*refs):
        if remap:
            (src_h, dst_h, ni_h, srco_h, dsto_h, degp_h,
             ni_v, src_v, dst_v, idx_v, val_v, zbuf, bounce_v, deg_sh) = refs
        else:
            (src_h, dst_h, degp_h,
             src_v, dst_v, idx_v, val_v, zbuf, bounce_v, deg_sh) = refs

        c, s, w = _worker_ids()
        base_e = w * EW
        pltpu.sync_copy(src_h.at[pl.ds(base_e, EW)], src_v)
        pltpu.sync_copy(dst_h.at[pl.ds(base_e, EW)], dst_v)
        if remap:
            pltpu.sync_copy(ni_h, ni_v)
        _fill_zbuf(zbuf)
        _zero_flat(zbuf, deg_sh, s * epw, epw)
        plsc.subcore_barrier()

        lane = lax.iota(jnp.int32, 16)
        ones = jnp.full((16,), 1.0, jnp.float32)
        zerosf = jnp.zeros((16,), jnp.float32)

        def blk(b, carry):
            off = b * 128
            for ch in range(8):
                o = off + ch * 16
                sv = src_v[pl.ds(o, 16)]
                dv = dst_v[pl.ds(o, 16)]
                if remap:
                    sv = jnp.where(sv >= 0,
                                   plsc.load_gather(ni_v, [jnp.maximum(sv, 0)]),
                                   -1)
                    dv = jnp.where(dv >= 0,
                                   plsc.load_gather(ni_v, [jnp.maximum(dv, 0)]),
                                   -1)
                    src_v[pl.ds(o, 16)] = sv
                    dst_v[pl.ds(o, 16)] = dv
                valid = (sv >= 0) & (dv >= 0)
                spread = Np + ((o + lane) & (TRASH - 1))
                idx_v[0, pl.ds(ch * 16, 16)] = jnp.where(valid, dv, spread)
                val_v[0, pl.ds(ch * 16, 16)] = jnp.where(valid, ones, zerosf)
            pltpu.sync_copy(val_v.at[0], deg_sh.at[idx_v.at[0]], add=True)
            return carry

        lax.fori_loop(0, NBLK, blk, 0)

        if remap:
            pltpu.sync_copy(src_v, srco_h.at[pl.ds(base_e, EW)])
            pltpu.sync_copy(dst_v, dsto_h.at[pl.ds(base_e, EW)])
        plsc.subcore_barrier()
        pltpu.sync_copy(deg_sh.at[pl.ds(s * epw, epw)], bounce_v)
        pltpu.sync_copy(bounce_v,
                        degp_h.at[pl.ds(c * acc_rows + s * epw, epw)])

    return pass_a


def _make_pass_b(Np):
    acc_rows = Np + TRASH
    rpw = acc_rows // 16
    gmask = 2047

    @functools.partial(
        pl.kernel,
        out_type=[jax.ShapeDtypeStruct((2, acc_rows, D), jnp.float32),
                  jax.ShapeDtypeStruct((2 * acc_rows,), jnp.float32)],
        scratch_types=[
            pltpu.VMEM((1, 128), jnp.int32),
            pltpu.VMEM((1, 128), jnp.int32),
            pltpu.VMEM((Np,), jnp.float32),
            pltpu.VMEM((1, 128), jnp.int32),
            pltpu.VMEM((1, 128), jnp.int32),
            pltpu.VMEM((1, 128), jnp.float32),
            pltpu.VMEM((128, D), jnp.float32),
            pltpu.VMEM((16, 128), jnp.float32),
            pltpu.VMEM((rpw,), jnp.float32),
            pltpu.SemaphoreType.DMA,
            pltpu.VMEM_SHARED((acc_rows, D), jnp.float32),
            pltpu.VMEM_SHARED((acc_rows,), jnp.float32),
        ],
        **_MESH)
    def pass_b(src_h, dst_h, xs_h, dinv_h, accp_h, sdegp_h,
               src_v, dst_v, dinv_v, gidx_v, didx_v, sval_v, rows_v,
               zbuf, bounce_v, sem, acc_sh, sdeg_sh):
        c, s, w = _worker_ids()
        base_e = w * EW
        pltpu.sync_copy(dinv_h, dinv_v)
        _fill_zbuf(zbuf)
        _zero_rows(zbuf, acc_sh, s * rpw, rpw)
        _zero_flat(zbuf, sdeg_sh, s * rpw, rpw)
        plsc.subcore_barrier()

        lane = lax.iota(jnp.int32, 16)
        zerosf = jnp.zeros((16,), jnp.float32)

        def blk(b, carry):
            eoff = base_e + b * 128
            pltpu.sync_copy(src_h.at[pl.ds(eoff, 128)], src_v.at[0])
            pltpu.sync_copy(dst_h.at[pl.ds(eoff, 128)], dst_v.at[0])
            for ch in range(8):
                o = b * 128 + ch * 16
                sv = src_v[0, pl.ds(ch * 16, 16)]
                dv = dst_v[0, pl.ds(ch * 16, 16)]
                valid = (sv >= 0) & (dv >= 0)
                pos = o + lane
                gidx_v[0, pl.ds(ch * 16, 16)] = jnp.where(valid, jnp.maximum(sv, 0),
                                                          pos & gmask)
                didx_v[0, pl.ds(ch * 16, 16)] = jnp.where(valid, dv,
                                                          Np + (pos & (TRASH - 1)))
                dsv = plsc.load_gather(dinv_v, [jnp.maximum(sv, 0)])
                sval_v[0, pl.ds(ch * 16, 16)] = jnp.where(valid, dsv, zerosf)
            pltpu.async_copy(xs_h.at[gidx_v.at[0]], rows_v, sem).wait()
            pltpu.sync_copy(rows_v, acc_sh.at[didx_v.at[0]], add=True)
            pltpu.sync_copy(sval_v.at[0], sdeg_sh.at[didx_v.at[0]], add=True)
            return carry

        lax.fori_loop(0, NBLK, blk, 0)

        plsc.subcore_barrier()
        nfull, rem = rpw // 128, rpw % 128
        for r in range(nfull):
            r0 = s * rpw + r * 128
            pltpu.sync_copy(acc_sh.at[pl.ds(r0, 128), :], rows_v)
            pltpu.sync_copy(rows_v, accp_h.at[c, pl.ds(r0, 128), :])
        if rem:
            r0 = s * rpw + nfull * 128
            pltpu.sync_copy(acc_sh.at[pl.ds(r0, rem), :],
                            rows_v.at[pl.ds(0, rem), :])
            pltpu.sync_copy(rows_v.at[pl.ds(0, rem), :],
                            accp_h.at[c, pl.ds(r0, rem), :])
        pltpu.sync_copy(sdeg_sh.at[pl.ds(s * rpw, rpw)], bounce_v)
        pltpu.sync_copy(bounce_v,
                        sdegp_h.at[pl.ds(c * acc_rows + s * rpw, rpw)])

    return pass_b


def _make_pass_c(Np, out_real):
    rpw = Np // NW
    scb = 80
    nb = rpw // scb
    out_rows = out_real + TRASH

    @functools.partial(
        pl.kernel,
        out_type=[jax.ShapeDtypeStruct((out_rows, D), jnp.float32)],
        scratch_types=[
            pltpu.VMEM((rpw,), jnp.int32),
            pltpu.VMEM((rpw, D), jnp.float32),
            pltpu.VMEM((1, scb), jnp.int32),
        ],
        **_MESH)
    def pass_c(hs_h, ni_h, xnew_h, ni_v, rows_v, idx_v):
        c, s, w = _worker_ids()
        base = w * rpw
        pltpu.sync_copy(ni_h.at[pl.ds(base, rpw)], ni_v)
        pltpu.sync_copy(hs_h.at[pl.ds(base, rpw), :], rows_v)
        lane = lax.iota(jnp.int32, 16)
        for b in range(nb):
            for ch in range(scb // 16):
                o = b * scb + ch * 16
                ni = ni_v[pl.ds(o, 16)]
                spread = out_real + ((o + lane) & (TRASH - 1))
                idx_v[0, pl.ds(ch * 16, 16)] = jnp.where(ni >= 0, ni, spread)
            pltpu.sync_copy(rows_v.at[pl.ds(b * scb, scb), :],
                            xnew_h.at[idx_v.at[0]])

    return pass_c


_RB = 512


def _tc_prep(x, Wbd, d0c, d1c, Np):
    nb = Np // _RB

    def body(x_ref, w_ref, d0_ref, d1_ref, xs_ref, dinv_ref):
        deg = d0_ref[...] + d1_ref[...] + 1.0
        dinv = lax.rsqrt(deg)
        xw = jnp.dot(x_ref[...], w_ref[...], preferred_element_type=jnp.float32)
        xs_ref[...] = xw * dinv
        dinv_ref[...] = dinv

    return pl.pallas_call(
        body,
        grid=(nb,),
        in_specs=[
            pl.BlockSpec((_RB, D), lambda i: (i, 0)),
            pl.BlockSpec((D, D), lambda i: (0, 0)),
            pl.BlockSpec((_RB, 1), lambda i: (i, 0)),
            pl.BlockSpec((_RB, 1), lambda i: (i, 0)),
        ],
        out_specs=[
            pl.BlockSpec((_RB, D), lambda i: (i, 0)),
            pl.BlockSpec((_RB, 1), lambda i: (i, 0)),
        ],
        out_shape=[jax.ShapeDtypeStruct((Np, D), jnp.float32),
                   jax.ShapeDtypeStruct((Np, 1), jnp.float32)],
    )(x, Wbd, d0c, d1c)


def _tc_post(acc_p, xs, dinv_c, sd0c, sd1c, b_pad, sparams, Np):
    nb = Np // _RB

    def body(a0_ref, a1_ref, xs_ref, dinv_ref, sd0_ref, sd1_ref, b_ref,
             p_ref, hs_ref, score_ref):
        acc = a0_ref[0] + a1_ref[0] + xs_ref[...]
        h = jnp.maximum(dinv_ref[...] * acc + b_ref[0:1, :], 0.0)
        sd = sd0_ref[...] + sd1_ref[...]
        dinv = dinv_ref[...]
        sw = p_ref[0, 0]
        sb = p_ref[0, 1]
        score = sw * (dinv * sd + dinv * dinv) + sb
        hs_ref[...] = h * jnp.tanh(score)
        score_ref[...] = score

    return pl.pallas_call(
        body,
        grid=(nb,),
        in_specs=[
            pl.BlockSpec((1, _RB, D), lambda i: (0, i, 0)),
            pl.BlockSpec((1, _RB, D), lambda i: (1, i, 0)),
            pl.BlockSpec((_RB, D), lambda i: (i, 0)),
            pl.BlockSpec((_RB, 1), lambda i: (i, 0)),
            pl.BlockSpec((_RB, 1), lambda i: (i, 0)),
            pl.BlockSpec((_RB, 1), lambda i: (i, 0)),
            pl.BlockSpec((8, 128), lambda i: (0, 0)),
            pl.BlockSpec((8, 128), lambda i: (0, 0)),
        ],
        out_specs=[
            pl.BlockSpec((_RB, D), lambda i: (i, 0)),
            pl.BlockSpec((_RB, 1), lambda i: (i, 0)),
        ],
        out_shape=[jax.ShapeDtypeStruct((Np, D), jnp.float32),
                   jax.ShapeDtypeStruct((Np, 1), jnp.float32)],
    )(acc_p, acc_p, xs, dinv_c, sd0c, sd1c, b_pad, sparams)


def _tc_topk(s_row, s_col, n_real, k, kp_next, L):
    NEG = -3e38

    def body(sr_ref, sc_ref, ni_ref):
        g = pl.program_id(0)
        srow = sr_ref[0, 0, :]
        scol = sc_ref[0, :, :]
        li = lax.broadcasted_iota(jnp.int32, (L,), 0)
        real_r = li < n_real
        srow = jnp.where(real_r, srow, NEG)
        cj = lax.broadcasted_iota(jnp.int32, (L, 1), 0)
        real_c = cj < n_real
        scol = jnp.where(real_c, scol, NEG)
        jj = lax.broadcasted_iota(jnp.int32, (L, L), 0)
        ii = lax.broadcasted_iota(jnp.int32, (L, L), 1)
        beats = (scol > srow) | ((scol == srow) & (jj < ii))
        cnt = jnp.sum(beats.astype(jnp.float32), axis=0).astype(jnp.int32)
        sel = real_r & (cnt < k)
        ni_ref[0, 0, :] = jnp.where(sel, g * kp_next + cnt, -1)

    return pl.pallas_call(
        body,
        grid=(G,),
        in_specs=[
            pl.BlockSpec((1, 1, L), lambda g: (g, 0, 0)),
            pl.BlockSpec((1, L, 1), lambda g: (g, 0, 0)),
        ],
        out_specs=pl.BlockSpec((1, 1, L), lambda g: (g, 0, 0)),
        out_shape=jax.ShapeDtypeStruct((G, 1, L), jnp.int32),
    )(s_row, s_col)


def _tc_final(x1, x2, x3, lin1_W, lin1_b, lin2_W, lin2_b, lin3_W, lin3_b):
    def body(x1_ref, x2_ref, x3_ref, w1_ref, b1_ref, w2_ref, b2_ref,
             w3_ref, b3_ref, out_ref):
        def readout(ref, k, kp):
            mx, mn = [], []
            for g in range(G):
                xg = ref[g * kp:g * kp + k, :]
                mx.append(jnp.max(xg, axis=0, keepdims=True))
                mn.append(jnp.sum(xg, axis=0, keepdims=True) / k)
            return jnp.concatenate(
                [jnp.concatenate(mx, axis=0), jnp.concatenate(mn, axis=0)],
                axis=1)

        r = (readout(x1_ref, 625, 640) + readout(x2_ref, 313, 320)
             + readout(x3_ref, 157, 160))
        h = jnp.maximum(jnp.dot(r, w1_ref[...],
                                preferred_element_type=jnp.float32)
                        + b1_ref[0:1, :], 0.0)
        h = jnp.maximum(jnp.dot(h, w2_ref[...],
                                preferred_element_type=jnp.float32)
                        + b2_ref[0:1, :64], 0.0)
        z = jnp.dot(h, w3_ref[...], preferred_element_type=jnp.float32) \
            + b3_ref[0:1, :10]
        m = jnp.max(z, axis=1, keepdims=True)
        e = jnp.exp(z - m)
        out_ref[...] = (z - m) - jnp.log(jnp.sum(e, axis=1, keepdims=True))

    return pl.pallas_call(
        body,
        out_shape=jax.ShapeDtypeStruct((G, 10), jnp.float32),
    )(x1, x2, x3, lin1_W, lin1_b, lin2_W, lin2_b, lin3_W, lin3_b)


def _blockdiag(W):
    H, di, do = W.shape
    M = jnp.zeros((H * di, H * do), jnp.float32)
    for i in range(H):
        M = M.at[i * di:(i + 1) * di, i * do:(i + 1) * do].set(W[i])
    return M


def _pad128(v):
    return jnp.pad(jnp.broadcast_to(v.reshape(1, -1), (8, v.shape[-1])),
                   ((0, 0), (0, 128 - v.shape[-1])))


def kernel(x, edge_index, batch,
           conv_W1, conv_b1, att_W1, att_b1, score_W1, score_b1,
           conv_W2, conv_b2, att_W2, att_b2, score_W2, score_b2,
           conv_W3, conv_b3, att_W3, att_b3, score_W3, score_b3,
           lin1_W, lin1_b, lin2_W, lin2_b, lin3_W, lin3_b):
    src = jnp.pad(edge_index[0].astype(jnp.int32), (0, EP - E0),
                  constant_values=-1)
    dst = jnp.pad(edge_index[1].astype(jnp.int32), (0, EP - E0),
                  constant_values=-1)

    convs = ((conv_W1, conv_b1, score_W1, score_b1),
             (conv_W2, conv_b2, score_W2, score_b2),
             (conv_W3, conv_b3, score_W3, score_b3))

    xcur = jnp.pad(x, ((0, STAGES[0]["Np"] - N0), (0, 0)))
    ni_prev = None
    readouts = []

    for si, cfg in enumerate(STAGES):
        Np, stride, n_real = cfg["Np"], cfg["stride"], cfg["n_real"]
        k, kp_next, L = cfg["k"], cfg["kp_next"], cfg["L"]
        cW, cb, sW, sb = convs[si]

        if si == 0:
            pass_a = _make_pass_a(Np, False, 0)
            (deg_p,) = pass_a(src, dst)
        else:
            pass_a = _make_pass_a(Np, True, ni_prev.shape[0])
            src, dst, deg_p = pass_a(src, dst, ni_prev)

        acc_rows = Np + TRASH
        d0c = deg_p[:Np].reshape(Np, 1)
        d1c = deg_p[acc_rows:acc_rows + Np].reshape(Np, 1)
        Wbd = _blockdiag(cW)
        xs, dinv_c = _tc_prep(xcur, Wbd, d0c, d1c, Np)

        pass_b = _make_pass_b(Np)
        acc_p, sdeg_p = pass_b(src, dst, xs, dinv_c.reshape(Np))

        b_cat = jnp.concatenate([cb[i] for i in range(4)])
        sparams = _pad128(jnp.stack([sW[0, 0], sb[0]]))
        sd0c = sdeg_p[:Np].reshape(Np, 1)
        sd1c = sdeg_p[acc_rows:acc_rows + Np].reshape(Np, 1)
        hs, score_c = _tc_post(acc_p, xs, dinv_c, sd0c, sd1c, _pad128(b_cat),
                               sparams, Np)

        sflat = score_c.reshape(Np)
        if si == 0:
            sreal = sflat[:N0].reshape(G, stride)
            sreal = jnp.pad(sreal, ((0, 0), (0, L - stride)),
                            constant_values=-3e38)
        else:
            sreal = sflat.reshape(G, L)
        ni3 = _tc_topk(sreal.reshape(G, 1, L), sreal.reshape(G, L, 1),
                       n_real, k, kp_next, L)

        if si == 0:
            ni_flat = jnp.pad(ni3[:, 0, :stride].reshape(-1),
                              (0, Np - N0), constant_values=-1)
        else:
            ni_flat = ni3[:, 0, :].reshape(-1)

        out_real = G * kp_next
        pass_c = _make_pass_c(Np, out_real)
        (x_new,) = pass_c(hs, ni_flat)

        readouts.append(x_new)
        ni_prev = ni_flat
        xcur = x_new[:out_real]

    out = _tc_final(readouts[0], readouts[1], readouts[2],
                    lin1_W, _pad128(lin1_b), lin2_W, _pad128(lin2_b),
                    lin3_W, _pad128(lin3_b))
    return out


# speedup vs baseline: 119.5975x; 1.2195x over previous
import functools
import math

import jax
import jax.numpy as jnp
from jax import lax
from jax.experimental import pallas as pl
from jax.experimental.pallas import tpu as pltpu
from jax.experimental.pallas import tpu_sc as plsc

N0 = 10000
E0 = 320000
D = 128
G = 8
NW = 32
EP = 327680
EW = EP // NW
EB = 80
NBE = EW // EB
TRASH = 128

STAGES = (
    dict(Np=10240, stride=1250, n_real=1250, k=625, kp_next=640, L=1280),
    dict(Np=5120, stride=640, n_real=625, k=313, kp_next=320, L=640),
    dict(Np=2560, stride=320, n_real=313, k=157, kp_next=160, L=320),
)

_MESH = dict(
    mesh=plsc.VectorSubcoreMesh(core_axis_name="c", subcore_axis_name="s"),
    compiler_params=pltpu.CompilerParams(
        use_tc_tiling_on_sc=False, needs_layout_passes=False),
)


def _worker_ids():
    c = lax.axis_index("c")
    s = lax.axis_index("s")
    return c, s, s * 2 + c


def _zero_rows(zbuf, shared, row0, nrows):
    nfull, rem = nrows // 16, nrows % 16

    def zrow(r, carry):
        pltpu.sync_copy(zbuf, shared.at[pl.ds(row0 + r * 16, 16), :])
        return carry

    lax.fori_loop(0, nfull, zrow, 0)
    if rem:
        pltpu.sync_copy(zbuf.at[pl.ds(0, rem), :],
                        shared.at[pl.ds(row0 + nfull * 16, rem), :])


def _zero_flat(zbuf, shared, off, n):
    nfull, rem = n // 128, n % 128
    for r in range(nfull):
        pltpu.sync_copy(zbuf.at[0, :], shared.at[pl.ds(off + r * 128, 128)])
    if rem:
        pltpu.sync_copy(zbuf.at[0, pl.ds(0, rem)],
                        shared.at[pl.ds(off + nfull * 128, rem)])


def _fill_zbuf(zbuf):
    zeros = jnp.zeros((16,), jnp.float32)

    def body(r, carry):
        for c in range(8):
            zbuf[r, pl.ds(c * 16, 16)] = zeros
        return carry

    lax.fori_loop(0, 16, body, 0)


def _make_pass_a(Np, remap, ni_len):
    acc_rows = Np + TRASH
    epw = acc_rows // 16

    out_type = [jax.ShapeDtypeStruct((2 * acc_rows,), jnp.float32)]
    if remap:
        out_type = [jax.ShapeDtypeStruct((2 * EP,), jnp.int32)] + out_type

    scratch = [
        pltpu.VMEM((2 * EW,), jnp.int32),
        pltpu.VMEM((1, EB), jnp.int32),
        pltpu.VMEM((1, EB), jnp.float32),
        pltpu.VMEM((16, 128), jnp.float32),
        pltpu.VMEM((epw,), jnp.float32),
    ]
    if remap:
        scratch = [pltpu.VMEM((ni_len,), jnp.int32)] + scratch

    shared = [pltpu.VMEM_SHARED((acc_rows,), jnp.float32)]

    @functools.partial(pl.kernel, out_type=out_type,
                       scratch_types=scratch + shared, **_MESH)
    def pass_a(*refs):
        if remap:
            (pck_h, ni_h, pcko_h, degp_h,
             ni_v, pck_v, idx_v, val_v, zbuf, bounce_v, deg_sh) = refs
        else:
            (pck_h, degp_h,
             pck_v, idx_v, val_v, zbuf, bounce_v, deg_sh) = refs

        c, s, w = _worker_ids()
        base_e = w * 2 * EW
        pltpu.sync_copy(pck_h.at[pl.ds(base_e, 2 * EW)], pck_v)
        if remap:
            pltpu.sync_copy(ni_h, ni_v)
        _fill_zbuf(zbuf)
        _zero_flat(zbuf, deg_sh, s * epw, epw)
        plsc.subcore_barrier()

        lane = lax.iota(jnp.int32, 16)
        ones = jnp.full((16,), 1.0, jnp.float32)
        zerosf = jnp.zeros((16,), jnp.float32)

        def blk(b, carry):
            off = b * 2 * EB
            for ch in range(EB // 16):
                so = off + ch * 16
                do = off + EB + ch * 16
                sv = pck_v[pl.ds(so, 16)]
                dv = pck_v[pl.ds(do, 16)]
                if remap:
                    sv = jnp.where(sv >= 0,
                                   plsc.load_gather(ni_v, [jnp.maximum(sv, 0)]),
                                   -1)
                    dv = jnp.where(dv >= 0,
                                   plsc.load_gather(ni_v, [jnp.maximum(dv, 0)]),
                                   -1)
                    pck_v[pl.ds(so, 16)] = sv
                    pck_v[pl.ds(do, 16)] = dv
                valid = (sv >= 0) & (dv >= 0)
                spread = Np + ((b * EB + ch * 16 + lane) & (TRASH - 1))
                idx_v[0, pl.ds(ch * 16, 16)] = jnp.where(valid, dv, spread)
                val_v[0, pl.ds(ch * 16, 16)] = jnp.where(valid, ones, zerosf)
            pltpu.sync_copy(val_v.at[0], deg_sh.at[idx_v.at[0]], add=True)
            return carry

        lax.fori_loop(0, NBE, blk, 0)

        if remap:
            pltpu.sync_copy(pck_v, pcko_h.at[pl.ds(base_e, 2 * EW)])
        plsc.subcore_barrier()
        pltpu.sync_copy(deg_sh.at[pl.ds(s * epw, epw)], bounce_v)
        pltpu.sync_copy(bounce_v,
                        degp_h.at[pl.ds(c * acc_rows + s * epw, epw)])

    return pass_a


def _make_pass_b(Np):
    acc_rows = Np + TRASH
    rpw = acc_rows // 16
    gmask = 2047

    @functools.partial(
        pl.kernel,
        out_type=[jax.ShapeDtypeStruct((2, acc_rows, D), jnp.float32),
                  jax.ShapeDtypeStruct((2 * acc_rows,), jnp.float32)],
        scratch_types=[
            pltpu.VMEM((2, 2 * EB), jnp.int32),
            pltpu.VMEM((Np,), jnp.float32),
            pltpu.VMEM((2, EB), jnp.int32),
            pltpu.VMEM((2, EB), jnp.int32),
            pltpu.VMEM((2, EB), jnp.float32),
            pltpu.VMEM((2, EB, D), jnp.float32),
            pltpu.VMEM((16, 128), jnp.float32),
            pltpu.VMEM((rpw,), jnp.float32),
            pltpu.SemaphoreType.DMA,
            pltpu.SemaphoreType.DMA,
            pltpu.VMEM_SHARED((acc_rows, D), jnp.float32),
            pltpu.VMEM_SHARED((acc_rows,), jnp.float32),
        ],
        **_MESH)
    def pass_b(pck_h, xs_h, dinv_h, accp_h, sdegp_h,
               pck_v, dinv_v, gidx_v, didx_v, sval_v, rows_v,
               zbuf, bounce_v, sem0, sem1, acc_sh, sdeg_sh):
        c, s, w = _worker_ids()
        base_e = w * 2 * EW
        pltpu.sync_copy(dinv_h, dinv_v)
        _fill_zbuf(zbuf)
        _zero_rows(zbuf, acc_sh, s * rpw, rpw)
        _zero_flat(zbuf, sdeg_sh, s * rpw, rpw)
        plsc.subcore_barrier()

        lane = lax.iota(jnp.int32, 16)
        zerosf = jnp.zeros((16,), jnp.float32)
        sems = (sem0, sem1)

        def blk(i, carry):
            handles = []
            for p in range(2):
                b = 2 * i + p
                eoff = base_e + b * 2 * EB
                pltpu.sync_copy(pck_h.at[pl.ds(eoff, 2 * EB)], pck_v.at[p])
                for ch in range(EB // 16):
                    o = b * EB + ch * 16
                    sv = pck_v[p, pl.ds(ch * 16, 16)]
                    dv = pck_v[p, pl.ds(EB + ch * 16, 16)]
                    valid = (sv >= 0) & (dv >= 0)
                    pos = o + lane
                    gidx_v[p, pl.ds(ch * 16, 16)] = jnp.where(
                        valid, jnp.maximum(sv, 0), pos & gmask)
                    didx_v[p, pl.ds(ch * 16, 16)] = jnp.where(
                        valid, dv, Np + (pos & (TRASH - 1)))
                    dsv = plsc.load_gather(dinv_v, [jnp.maximum(sv, 0)])
                    sval_v[p, pl.ds(ch * 16, 16)] = jnp.where(valid, dsv, zerosf)
                handles.append(
                    pltpu.async_copy(xs_h.at[gidx_v.at[p]], rows_v.at[p],
                                     sems[p]))
            for p in range(2):
                handles[p].wait()
                pltpu.sync_copy(rows_v.at[p], acc_sh.at[didx_v.at[p]], add=True)
                pltpu.sync_copy(sval_v.at[p], sdeg_sh.at[didx_v.at[p]], add=True)
            return carry

        lax.fori_loop(0, NBE // 2, blk, 0)

        plsc.subcore_barrier()
        nfull, rem = rpw // EB, rpw % EB
        for r in range(nfull):
            r0 = s * rpw + r * EB
            pltpu.sync_copy(acc_sh.at[pl.ds(r0, EB), :], rows_v.at[r % 2])
            pltpu.sync_copy(rows_v.at[r % 2], accp_h.at[c, pl.ds(r0, EB), :])
        if rem:
            r0 = s * rpw + nfull * EB
            pltpu.sync_copy(acc_sh.at[pl.ds(r0, rem), :],
                            rows_v.at[0, pl.ds(0, rem), :])
            pltpu.sync_copy(rows_v.at[0, pl.ds(0, rem), :],
                            accp_h.at[c, pl.ds(r0, rem), :])
        pltpu.sync_copy(sdeg_sh.at[pl.ds(s * rpw, rpw)], bounce_v)
        pltpu.sync_copy(bounce_v,
                        sdegp_h.at[pl.ds(c * acc_rows + s * rpw, rpw)])

    return pass_b


def _make_pass_c(Np, out_real):
    rpw = Np // NW
    scb = 80
    nb = rpw // scb
    out_rows = out_real + TRASH

    @functools.partial(
        pl.kernel,
        out_type=[jax.ShapeDtypeStruct((out_rows, D), jnp.float32)],
        scratch_types=[
            pltpu.VMEM((rpw,), jnp.int32),
            pltpu.VMEM((rpw, D), jnp.float32),
            pltpu.VMEM((1, scb), jnp.int32),
        ],
        **_MESH)
    def pass_c(hs_h, ni_h, xnew_h, ni_v, rows_v, idx_v):
        c, s, w = _worker_ids()
        base = w * rpw
        pltpu.sync_copy(ni_h.at[pl.ds(base, rpw)], ni_v)
        pltpu.sync_copy(hs_h.at[pl.ds(base, rpw), :], rows_v)
        lane = lax.iota(jnp.int32, 16)
        for b in range(nb):
            for ch in range(scb // 16):
                o = b * scb + ch * 16
                ni = ni_v[pl.ds(o, 16)]
                spread = out_real + ((o + lane) & (TRASH - 1))
                idx_v[0, pl.ds(ch * 16, 16)] = jnp.where(ni >= 0, ni, spread)
            pltpu.sync_copy(rows_v.at[pl.ds(b * scb, scb), :],
                            xnew_h.at[idx_v.at[0]])

    return pass_c


_RB = 512


def _tc_prep(x, Wbd, d0c, d1c, Np):
    nb = Np // _RB

    def body(x_ref, w_ref, d0_ref, d1_ref, xs_ref, dinv_ref):
        deg = d0_ref[...] + d1_ref[...] + 1.0
        dinv = lax.rsqrt(deg)
        xw = jnp.dot(x_ref[...], w_ref[...], preferred_element_type=jnp.float32)
        xs_ref[...] = xw * dinv
        dinv_ref[...] = dinv

    return pl.pallas_call(
        body,
        grid=(nb,),
        in_specs=[
            pl.BlockSpec((_RB, D), lambda i: (i, 0)),
            pl.BlockSpec((D, D), lambda i: (0, 0)),
            pl.BlockSpec((_RB, 1), lambda i: (i, 0)),
            pl.BlockSpec((_RB, 1), lambda i: (i, 0)),
        ],
        out_specs=[
            pl.BlockSpec((_RB, D), lambda i: (i, 0)),
            pl.BlockSpec((_RB, 1), lambda i: (i, 0)),
        ],
        out_shape=[jax.ShapeDtypeStruct((Np, D), jnp.float32),
                   jax.ShapeDtypeStruct((Np, 1), jnp.float32)],
    )(x, Wbd, d0c, d1c)


def _tc_post(acc_p, xs, dinv_c, sd0c, sd1c, b_pad, sparams, Np):
    nb = Np // _RB

    def body(a0_ref, a1_ref, xs_ref, dinv_ref, sd0_ref, sd1_ref, b_ref,
             p_ref, hs_ref, score_ref):
        acc = a0_ref[0] + a1_ref[0] + xs_ref[...]
        h = jnp.maximum(dinv_ref[...] * acc + b_ref[0:1, :], 0.0)
        sd = sd0_ref[...] + sd1_ref[...]
        dinv = dinv_ref[...]
        sw = p_ref[0, 0]
        sb = p_ref[0, 1]
        score = sw * (dinv * sd + dinv * dinv) + sb
        hs_ref[...] = h * jnp.tanh(score)
        score_ref[...] = score

    return pl.pallas_call(
        body,
        grid=(nb,),
        in_specs=[
            pl.BlockSpec((1, _RB, D), lambda i: (0, i, 0)),
            pl.BlockSpec((1, _RB, D), lambda i: (1, i, 0)),
            pl.BlockSpec((_RB, D), lambda i: (i, 0)),
            pl.BlockSpec((_RB, 1), lambda i: (i, 0)),
            pl.BlockSpec((_RB, 1), lambda i: (i, 0)),
            pl.BlockSpec((_RB, 1), lambda i: (i, 0)),
            pl.BlockSpec((8, 128), lambda i: (0, 0)),
            pl.BlockSpec((8, 128), lambda i: (0, 0)),
        ],
        out_specs=[
            pl.BlockSpec((_RB, D), lambda i: (i, 0)),
            pl.BlockSpec((_RB, 1), lambda i: (i, 0)),
        ],
        out_shape=[jax.ShapeDtypeStruct((Np, D), jnp.float32),
                   jax.ShapeDtypeStruct((Np, 1), jnp.float32)],
    )(acc_p, acc_p, xs, dinv_c, sd0c, sd1c, b_pad, sparams)


def _tc_topk(s_row, s_col, n_real, k, kp_next, L):
    NEG = -3e38

    def body(sr_ref, sc_ref, ni_ref):
        g = pl.program_id(0)
        srow = sr_ref[0, 0, :]
        scol = sc_ref[0, :, :]
        li = lax.broadcasted_iota(jnp.int32, (L,), 0)
        real_r = li < n_real
        srow = jnp.where(real_r, srow, NEG)
        cj = lax.broadcasted_iota(jnp.int32, (L, 1), 0)
        real_c = cj < n_real
        scol = jnp.where(real_c, scol, NEG)
        jj = lax.broadcasted_iota(jnp.int32, (L, L), 0)
        ii = lax.broadcasted_iota(jnp.int32, (L, L), 1)
        beats = (scol > srow) | ((scol == srow) & (jj < ii))
        cnt = jnp.sum(beats.astype(jnp.float32), axis=0).astype(jnp.int32)
        sel = real_r & (cnt < k)
        ni_ref[0, 0, :] = jnp.where(sel, g * kp_next + cnt, -1)

    return pl.pallas_call(
        body,
        grid=(G,),
        in_specs=[
            pl.BlockSpec((1, 1, L), lambda g: (g, 0, 0)),
            pl.BlockSpec((1, L, 1), lambda g: (g, 0, 0)),
        ],
        out_specs=pl.BlockSpec((1, 1, L), lambda g: (g, 0, 0)),
        out_shape=jax.ShapeDtypeStruct((G, 1, L), jnp.int32),
    )(s_row, s_col)


def _tc_final(x1, x2, x3, lin1_W, lin1_b, lin2_W, lin2_b, lin3_W, lin3_b):
    def body(x1_ref, x2_ref, x3_ref, w1_ref, b1_ref, w2_ref, b2_ref,
             w3_ref, b3_ref, out_ref):
        def readout(ref, k, kp):
            mx, mn = [], []
            for g in range(G):
                xg = ref[g * kp:g * kp + k, :]
                mx.append(jnp.max(xg, axis=0, keepdims=True))
                mn.append(jnp.sum(xg, axis=0, keepdims=True) / k)
            return jnp.concatenate(
                [jnp.concatenate(mx, axis=0), jnp.concatenate(mn, axis=0)],
                axis=1)

        r = (readout(x1_ref, 625, 640) + readout(x2_ref, 313, 320)
             + readout(x3_ref, 157, 160))
        h = jnp.maximum(jnp.dot(r, w1_ref[...],
                                preferred_element_type=jnp.float32)
                        + b1_ref[0:1, :], 0.0)
        h = jnp.maximum(jnp.dot(h, w2_ref[...],
                                preferred_element_type=jnp.float32)
                        + b2_ref[0:1, :64], 0.0)
        z = jnp.dot(h, w3_ref[...], preferred_element_type=jnp.float32) \
            + b3_ref[0:1, :10]
        m = jnp.max(z, axis=1, keepdims=True)
        e = jnp.exp(z - m)
        out_ref[...] = (z - m) - jnp.log(jnp.sum(e, axis=1, keepdims=True))

    return pl.pallas_call(
        body,
        out_shape=jax.ShapeDtypeStruct((G, 10), jnp.float32),
    )(x1, x2, x3, lin1_W, lin1_b, lin2_W, lin2_b, lin3_W, lin3_b)


def _blockdiag(W):
    H, di, do = W.shape
    M = jnp.zeros((H * di, H * do), jnp.float32)
    for i in range(H):
        M = M.at[i * di:(i + 1) * di, i * do:(i + 1) * do].set(W[i])
    return M


def _pad128(v):
    return jnp.pad(jnp.broadcast_to(v.reshape(1, -1), (8, v.shape[-1])),
                   ((0, 0), (0, 128 - v.shape[-1])))


def kernel(x, edge_index, batch,
           conv_W1, conv_b1, att_W1, att_b1, score_W1, score_b1,
           conv_W2, conv_b2, att_W2, att_b2, score_W2, score_b2,
           conv_W3, conv_b3, att_W3, att_b3, score_W3, score_b3,
           lin1_W, lin1_b, lin2_W, lin2_b, lin3_W, lin3_b):
    src = jnp.pad(edge_index[0].astype(jnp.int32), (0, EP - E0),
                  constant_values=-1)
    dst = jnp.pad(edge_index[1].astype(jnp.int32), (0, EP - E0),
                  constant_values=-1)
    pck = jnp.stack([src.reshape(NW, NBE, EB), dst.reshape(NW, NBE, EB)],
                    axis=2).reshape(-1)

    convs = ((conv_W1, conv_b1, score_W1, score_b1),
             (conv_W2, conv_b2, score_W2, score_b2),
             (conv_W3, conv_b3, score_W3, score_b3))

    xcur = jnp.pad(x, ((0, STAGES[0]["Np"] - N0), (0, 0)))
    ni_prev = None
    readouts = []

    for si, cfg in enumerate(STAGES):
        Np, stride, n_real = cfg["Np"], cfg["stride"], cfg["n_real"]
        k, kp_next, L = cfg["k"], cfg["kp_next"], cfg["L"]
        cW, cb, sW, sb = convs[si]

        if si == 0:
            pass_a = _make_pass_a(Np, False, 0)
            (deg_p,) = pass_a(pck)
        else:
            pass_a = _make_pass_a(Np, True, ni_prev.shape[0])
            pck, deg_p = pass_a(pck, ni_prev)

        acc_rows = Np + TRASH
        d0c = deg_p[:Np].reshape(Np, 1)
        d1c = deg_p[acc_rows:acc_rows + Np].reshape(Np, 1)
        Wbd = _blockdiag(cW)
        xs, dinv_c = _tc_prep(xcur, Wbd, d0c, d1c, Np)

        pass_b = _make_pass_b(Np)
        acc_p, sdeg_p = pass_b(pck, xs, dinv_c.reshape(Np))

        b_cat = jnp.concatenate([cb[i] for i in range(4)])
        sparams = _pad128(jnp.stack([sW[0, 0], sb[0]]))
        sd0c = sdeg_p[:Np].reshape(Np, 1)
        sd1c = sdeg_p[acc_rows:acc_rows + Np].reshape(Np, 1)
        hs, score_c = _tc_post(acc_p, xs, dinv_c, sd0c, sd1c, _pad128(b_cat),
                               sparams, Np)

        sflat = score_c.reshape(Np)
        if si == 0:
            sreal = sflat[:N0].reshape(G, stride)
            sreal = jnp.pad(sreal, ((0, 0), (0, L - stride)),
                            constant_values=-3e38)
        else:
            sreal = sflat.reshape(G, L)
        ni3 = _tc_topk(sreal.reshape(G, 1, L), sreal.reshape(G, L, 1),
                       n_real, k, kp_next, L)

        if si == 0:
            ni_flat = jnp.pad(ni3[:, 0, :stride].reshape(-1),
                              (0, Np - N0), constant_values=-1)
        else:
            ni_flat = ni3[:, 0, :].reshape(-1)

        out_real = G * kp_next
        pass_c = _make_pass_c(Np, out_real)
        (x_new,) = pass_c(hs, ni_flat)

        readouts.append(x_new)
        ni_prev = ni_flat
        xcur = x_new[:out_real]

    out = _tc_final(readouts[0], readouts[1], readouts[2],
                    lin1_W, _pad128(lin1_b), lin2_W, _pad128(lin2_b),
                    lin3_W, _pad128(lin3_b))
    return out


# speedup vs baseline: 167.1840x; 1.3979x over previous
import functools
import math

import jax
import jax.numpy as jnp
from jax import lax
from jax.experimental import pallas as pl
from jax.experimental.pallas import tpu as pltpu
from jax.experimental.pallas import tpu_sc as plsc

N0 = 10000
E0 = 320000
D = 128
G = 8
NW = 32
EP = 327680
EW = EP // NW
EB = 80
NBE = EW // EB
TRASH = 128

STAGES = (
    dict(Np=10240, stride=1250, n_real=1250, k=625, kp_next=640, L=1280),
    dict(Np=5120, stride=640, n_real=625, k=313, kp_next=320, L=640),
    dict(Np=2560, stride=320, n_real=313, k=157, kp_next=160, L=320),
)

_MESH = dict(
    mesh=plsc.VectorSubcoreMesh(core_axis_name="c", subcore_axis_name="s"),
    compiler_params=pltpu.CompilerParams(
        use_tc_tiling_on_sc=False, needs_layout_passes=False),
)


def _worker_ids():
    c = lax.axis_index("c")
    s = lax.axis_index("s")
    return c, s, s * 2 + c


def _zero_rows(zbuf, shared, row0, nrows):
    nfull, rem = nrows // 16, nrows % 16

    def zrow(r, carry):
        pltpu.sync_copy(zbuf, shared.at[pl.ds(row0 + r * 16, 16), :])
        return carry

    lax.fori_loop(0, nfull, zrow, 0)
    if rem:
        pltpu.sync_copy(zbuf.at[pl.ds(0, rem), :],
                        shared.at[pl.ds(row0 + nfull * 16, rem), :])


def _zero_flat(zbuf, shared, off, n):
    nfull, rem = n // 128, n % 128
    for r in range(nfull):
        pltpu.sync_copy(zbuf.at[0, :], shared.at[pl.ds(off + r * 128, 128)])
    if rem:
        pltpu.sync_copy(zbuf.at[0, pl.ds(0, rem)],
                        shared.at[pl.ds(off + nfull * 128, rem)])


def _fill_zbuf(zbuf):
    zeros = jnp.zeros((16,), jnp.float32)

    def body(r, carry):
        for c in range(8):
            zbuf[r, pl.ds(c * 16, 16)] = zeros
        return carry

    lax.fori_loop(0, 16, body, 0)


def _make_pass_a(Np, remap, ni_len):
    acc_rows = Np + TRASH
    epw = acc_rows // 16
    marg = EW + 2 * EB + 16

    out_type = [jax.ShapeDtypeStruct((2 * acc_rows,), jnp.float32)]
    if remap:
        out_type = [jax.ShapeDtypeStruct((2 * EP,), jnp.int32),
                    jax.ShapeDtypeStruct((NW * 16,), jnp.int32)] + out_type

    scratch = [
        pltpu.VMEM((EW,), jnp.int32),
        pltpu.VMEM((EW,), jnp.int32),
        pltpu.VMEM((1, EB), jnp.int32),
        pltpu.VMEM((1, EB), jnp.float32),
        pltpu.VMEM((16, 128), jnp.float32),
        pltpu.VMEM((epw,), jnp.float32),
    ]
    if remap:
        scratch = [pltpu.VMEM((ni_len,), jnp.int32),
                   pltpu.VMEM((marg,), jnp.int32),
                   pltpu.VMEM((marg,), jnp.int32),
                   pltpu.VMEM((16,), jnp.int32),
                   pltpu.VMEM((16,), jnp.int32)] + scratch

    shared = [pltpu.VMEM_SHARED((acc_rows,), jnp.float32)]

    @functools.partial(pl.kernel, out_type=out_type,
                       scratch_types=scratch + shared, **_MESH)
    def pass_a(*refs):
        if remap:
            (pck_h, ni_h, cntin_h, pcko_h, cnt_h, degp_h,
             ni_v, srcc_v, dstc_v, cnt_v, cntin_v,
             src_v, dst_v, idx_v, val_v, zbuf, bounce_v, deg_sh) = refs
        else:
            (pck_h, degp_h,
             src_v, dst_v, idx_v, val_v, zbuf, bounce_v, deg_sh) = refs

        c, s, w = _worker_ids()
        base_e = w * EW
        pltpu.sync_copy(pck_h.at[pl.ds(base_e, EW)], src_v)
        pltpu.sync_copy(pck_h.at[pl.ds(EP + base_e, EW)], dst_v)
        if remap:
            pltpu.sync_copy(ni_h, ni_v)
            pltpu.sync_copy(cntin_h.at[pl.ds(w * 16, 16)], cntin_v)
        _fill_zbuf(zbuf)
        _zero_flat(zbuf, deg_sh, s * epw, epw)
        plsc.subcore_barrier()

        lane = lax.iota(jnp.int32, 16)
        ones = jnp.full((16,), 1.0, jnp.float32)
        zerosf = jnp.zeros((16,), jnp.float32)

        def blk(b, cnt):
            for ch in range(EB // 16):
                o = b * EB + ch * 16
                sv = src_v[pl.ds(o, 16)]
                dv = dst_v[pl.ds(o, 16)]
                if remap:
                    sv = jnp.where(sv >= 0,
                                   plsc.load_gather(ni_v, [jnp.maximum(sv, 0)]),
                                   -1)
                    dv = jnp.where(dv >= 0,
                                   plsc.load_gather(ni_v, [jnp.maximum(dv, 0)]),
                                   -1)
                valid = (sv >= 0) & (dv >= 0)
                if remap:
                    plsc.store_compressed(srcc_v.at[pl.ds(cnt, 16)], sv,
                                          mask=valid)
                    plsc.store_compressed(dstc_v.at[pl.ds(cnt, 16)], dv,
                                          mask=valid)
                    cnt = cnt + jnp.sum(valid.astype(jnp.int32))
                spread = Np + ((b * EB + ch * 16 + lane) & (TRASH - 1))
                idx_v[0, pl.ds(ch * 16, 16)] = jnp.where(valid, dv, spread)
                val_v[0, pl.ds(ch * 16, 16)] = jnp.where(valid, ones, zerosf)
            pltpu.sync_copy(val_v.at[0], deg_sh.at[idx_v.at[0]], add=True)
            return cnt

        if remap:
            civ = cntin_v[pl.ds(0, 16)]
            nblk = (civ[0] + (EB - 1)) // EB
        else:
            nblk = NBE
        cnt = lax.fori_loop(0, nblk, blk, jnp.int32(0))

        if remap:
            neg = jnp.full((16,), -1, jnp.int32)
            for f in range(2 * EB // 16):
                srcc_v[pl.ds(cnt + f * 16, 16)] = neg
                dstc_v[pl.ds(cnt + f * 16, 16)] = neg
            pltpu.sync_copy(srcc_v.at[pl.ds(0, EW)],
                            pcko_h.at[pl.ds(base_e, EW)])
            pltpu.sync_copy(dstc_v.at[pl.ds(0, EW)],
                            pcko_h.at[pl.ds(EP + base_e, EW)])
            cnt_v[pl.ds(0, 16)] = jnp.full((16,), 1, jnp.int32) * cnt
            pltpu.sync_copy(cnt_v, cnt_h.at[pl.ds(w * 16, 16)])
        plsc.subcore_barrier()
        pltpu.sync_copy(deg_sh.at[pl.ds(s * epw, epw)], bounce_v)
        pltpu.sync_copy(bounce_v,
                        degp_h.at[pl.ds(c * acc_rows + s * epw, epw)])

    return pass_a


def _make_pass_b(Np, dynamic):
    acc_rows = Np + TRASH
    rpw = acc_rows // 16
    gmask = 2047

    @functools.partial(
        pl.kernel,
        out_type=[jax.ShapeDtypeStruct((2, acc_rows, D), jnp.float32),
                  jax.ShapeDtypeStruct((2 * acc_rows,), jnp.float32)],
        scratch_types=[
            pltpu.VMEM((2, 2 * EB), jnp.int32),
            pltpu.VMEM((Np,), jnp.float32),
            pltpu.VMEM((2, EB), jnp.int32),
            pltpu.VMEM((2, EB), jnp.int32),
            pltpu.VMEM((2, EB), jnp.float32),
            pltpu.VMEM((2, EB, D), jnp.float32),
            pltpu.VMEM((16,), jnp.int32),
            pltpu.VMEM((16, 128), jnp.float32),
            pltpu.VMEM((rpw,), jnp.float32),
            pltpu.SemaphoreType.DMA,
            pltpu.SemaphoreType.DMA,
            pltpu.VMEM_SHARED((acc_rows, D), jnp.float32),
            pltpu.VMEM_SHARED((acc_rows,), jnp.float32),
        ],
        **_MESH)
    def pass_b(*refs):
        if dynamic:
            (pck_h, cnt_h, xs_h, dinv_h, accp_h, sdegp_h,
             pck_v, dinv_v, gidx_v, didx_v, sval_v, rows_v, cnt_v,
             zbuf, bounce_v, sem0, sem1, acc_sh, sdeg_sh) = refs
        else:
            (pck_h, xs_h, dinv_h, accp_h, sdegp_h,
             pck_v, dinv_v, gidx_v, didx_v, sval_v, rows_v, cnt_v,
             zbuf, bounce_v, sem0, sem1, acc_sh, sdeg_sh) = refs
        c, s, w = _worker_ids()
        base_e = w * EW
        pltpu.sync_copy(dinv_h, dinv_v)
        if dynamic:
            pltpu.sync_copy(cnt_h.at[pl.ds(w * 16, 16)], cnt_v)
        _fill_zbuf(zbuf)
        _zero_rows(zbuf, acc_sh, s * rpw, rpw)
        _zero_flat(zbuf, sdeg_sh, s * rpw, rpw)
        plsc.subcore_barrier()

        lane = lax.iota(jnp.int32, 16)
        zerosf = jnp.zeros((16,), jnp.float32)
        sems = (sem0, sem1)

        def blk(i, carry):
            handles = []
            for p in range(2):
                b = 2 * i + p
                eoff = base_e + b * EB
                pltpu.sync_copy(pck_h.at[pl.ds(eoff, EB)],
                                pck_v.at[p, pl.ds(0, EB)])
                pltpu.sync_copy(pck_h.at[pl.ds(EP + eoff, EB)],
                                pck_v.at[p, pl.ds(EB, EB)])
                for ch in range(EB // 16):
                    o = b * EB + ch * 16
                    sv = pck_v[p, pl.ds(ch * 16, 16)]
                    dv = pck_v[p, pl.ds(EB + ch * 16, 16)]
                    valid = (sv >= 0) & (dv >= 0)
                    pos = o + lane
                    gidx_v[p, pl.ds(ch * 16, 16)] = jnp.where(
                        valid, jnp.maximum(sv, 0), pos & gmask)
                    didx_v[p, pl.ds(ch * 16, 16)] = jnp.where(
                        valid, dv, Np + (pos & (TRASH - 1)))
                    dsv = plsc.load_gather(dinv_v, [jnp.maximum(sv, 0)])
                    sval_v[p, pl.ds(ch * 16, 16)] = jnp.where(valid, dsv, zerosf)
                handles.append(
                    pltpu.async_copy(xs_h.at[gidx_v.at[p]], rows_v.at[p],
                                     sems[p]))
            for p in range(2):
                handles[p].wait()
                pltpu.sync_copy(rows_v.at[p], acc_sh.at[didx_v.at[p]], add=True)
                pltpu.sync_copy(sval_v.at[p], sdeg_sh.at[didx_v.at[p]], add=True)
            return carry

        if dynamic:
            cv = cnt_v[pl.ds(0, 16)]
            npairs = (cv[0] + (2 * EB - 1)) // (2 * EB)
            lax.fori_loop(0, npairs, blk, 0)
        else:
            lax.fori_loop(0, NBE // 2, blk, 0)

        plsc.subcore_barrier()
        nfull, rem = rpw // EB, rpw % EB
        for r in range(nfull):
            r0 = s * rpw + r * EB
            pltpu.sync_copy(acc_sh.at[pl.ds(r0, EB), :], rows_v.at[r % 2])
            pltpu.sync_copy(rows_v.at[r % 2], accp_h.at[c, pl.ds(r0, EB), :])
        if rem:
            r0 = s * rpw + nfull * EB
            pltpu.sync_copy(acc_sh.at[pl.ds(r0, rem), :],
                            rows_v.at[0, pl.ds(0, rem), :])
            pltpu.sync_copy(rows_v.at[0, pl.ds(0, rem), :],
                            accp_h.at[c, pl.ds(r0, rem), :])
        pltpu.sync_copy(sdeg_sh.at[pl.ds(s * rpw, rpw)], bounce_v)
        pltpu.sync_copy(bounce_v,
                        sdegp_h.at[pl.ds(c * acc_rows + s * rpw, rpw)])

    return pass_b


def _make_pass_c(Np, out_real):
    rpw = Np // NW
    scb = 80
    nb = rpw // scb
    out_rows = out_real + TRASH

    @functools.partial(
        pl.kernel,
        out_type=[jax.ShapeDtypeStruct((out_rows, D), jnp.float32)],
        scratch_types=[
            pltpu.VMEM((rpw,), jnp.int32),
            pltpu.VMEM((rpw, D), jnp.float32),
            pltpu.VMEM((1, scb), jnp.int32),
        ],
        **_MESH)
    def pass_c(hs_h, ni_h, xnew_h, ni_v, rows_v, idx_v):
        c, s, w = _worker_ids()
        base = w * rpw
        pltpu.sync_copy(ni_h.at[pl.ds(base, rpw)], ni_v)
        pltpu.sync_copy(hs_h.at[pl.ds(base, rpw), :], rows_v)
        lane = lax.iota(jnp.int32, 16)
        for b in range(nb):
            for ch in range(scb // 16):
                o = b * scb + ch * 16
                ni = ni_v[pl.ds(o, 16)]
                spread = out_real + ((o + lane) & (TRASH - 1))
                idx_v[0, pl.ds(ch * 16, 16)] = jnp.where(ni >= 0, ni, spread)
            pltpu.sync_copy(rows_v.at[pl.ds(b * scb, scb), :],
                            xnew_h.at[idx_v.at[0]])

    return pass_c


_RB = 512


def _tc_prep(x, Wbd, d0c, d1c, Np):
    nb = Np // _RB

    def body(x_ref, w_ref, d0_ref, d1_ref, xs_ref, dinv_ref):
        deg = d0_ref[...] + d1_ref[...] + 1.0
        dinv = lax.rsqrt(deg)
        xw = jnp.dot(x_ref[...], w_ref[...], preferred_element_type=jnp.float32)
        xs_ref[...] = xw * dinv
        dinv_ref[...] = dinv

    return pl.pallas_call(
        body,
        grid=(nb,),
        in_specs=[
            pl.BlockSpec((_RB, D), lambda i: (i, 0)),
            pl.BlockSpec((D, D), lambda i: (0, 0)),
            pl.BlockSpec((_RB, 1), lambda i: (i, 0)),
            pl.BlockSpec((_RB, 1), lambda i: (i, 0)),
        ],
        out_specs=[
            pl.BlockSpec((_RB, D), lambda i: (i, 0)),
            pl.BlockSpec((_RB, 1), lambda i: (i, 0)),
        ],
        out_shape=[jax.ShapeDtypeStruct((Np, D), jnp.float32),
                   jax.ShapeDtypeStruct((Np, 1), jnp.float32)],
    )(x, Wbd, d0c, d1c)


def _tc_post(acc_p, xs, dinv_c, sd0c, sd1c, b_pad, sparams, Np):
    nb = Np // _RB

    def body(a0_ref, a1_ref, xs_ref, dinv_ref, sd0_ref, sd1_ref, b_ref,
             p_ref, hs_ref, score_ref):
        acc = a0_ref[0] + a1_ref[0] + xs_ref[...]
        h = jnp.maximum(dinv_ref[...] * acc + b_ref[0:1, :], 0.0)
        sd = sd0_ref[...] + sd1_ref[...]
        dinv = dinv_ref[...]
        sw = p_ref[0, 0]
        sb = p_ref[0, 1]
        score = sw * (dinv * sd + dinv * dinv) + sb
        hs_ref[...] = h * jnp.tanh(score)
        score_ref[...] = score

    return pl.pallas_call(
        body,
        grid=(nb,),
        in_specs=[
            pl.BlockSpec((1, _RB, D), lambda i: (0, i, 0)),
            pl.BlockSpec((1, _RB, D), lambda i: (1, i, 0)),
            pl.BlockSpec((_RB, D), lambda i: (i, 0)),
            pl.BlockSpec((_RB, 1), lambda i: (i, 0)),
            pl.BlockSpec((_RB, 1), lambda i: (i, 0)),
            pl.BlockSpec((_RB, 1), lambda i: (i, 0)),
            pl.BlockSpec((8, 128), lambda i: (0, 0)),
            pl.BlockSpec((8, 128), lambda i: (0, 0)),
        ],
        out_specs=[
            pl.BlockSpec((_RB, D), lambda i: (i, 0)),
            pl.BlockSpec((_RB, 1), lambda i: (i, 0)),
        ],
        out_shape=[jax.ShapeDtypeStruct((Np, D), jnp.float32),
                   jax.ShapeDtypeStruct((Np, 1), jnp.float32)],
    )(acc_p, acc_p, xs, dinv_c, sd0c, sd1c, b_pad, sparams)


def _tc_topk(s_row, s_col, n_real, k, kp_next, L):
    NEG = -3e38

    def body(sr_ref, sc_ref, ni_ref):
        g = pl.program_id(0)
        srow = sr_ref[0, 0, :]
        scol = sc_ref[0, :, :]
        li = lax.broadcasted_iota(jnp.int32, (L,), 0)
        real_r = li < n_real
        srow = jnp.where(real_r, srow, NEG)
        cj = lax.broadcasted_iota(jnp.int32, (L, 1), 0)
        real_c = cj < n_real
        scol = jnp.where(real_c, scol, NEG)
        jj = lax.broadcasted_iota(jnp.int32, (L, L), 0)
        ii = lax.broadcasted_iota(jnp.int32, (L, L), 1)
        beats = (scol > srow) | ((scol == srow) & (jj < ii))
        cnt = jnp.sum(beats.astype(jnp.float32), axis=0).astype(jnp.int32)
        sel = real_r & (cnt < k)
        ni_ref[0, 0, :] = jnp.where(sel, g * kp_next + cnt, -1)

    return pl.pallas_call(
        body,
        grid=(G,),
        in_specs=[
            pl.BlockSpec((1, 1, L), lambda g: (g, 0, 0)),
            pl.BlockSpec((1, L, 1), lambda g: (g, 0, 0)),
        ],
        out_specs=pl.BlockSpec((1, 1, L), lambda g: (g, 0, 0)),
        out_shape=jax.ShapeDtypeStruct((G, 1, L), jnp.int32),
    )(s_row, s_col)


def _tc_final(x1, x2, x3, lin1_W, lin1_b, lin2_W, lin2_b, lin3_W, lin3_b):
    def body(x1_ref, x2_ref, x3_ref, w1_ref, b1_ref, w2_ref, b2_ref,
             w3_ref, b3_ref, out_ref):
        def readout(ref, k, kp):
            mx, mn = [], []
            for g in range(G):
                xg = ref[g * kp:g * kp + k, :]
                mx.append(jnp.max(xg, axis=0, keepdims=True))
                mn.append(jnp.sum(xg, axis=0, keepdims=True) / k)
            return jnp.concatenate(
                [jnp.concatenate(mx, axis=0), jnp.concatenate(mn, axis=0)],
                axis=1)

        r = (readout(x1_ref, 625, 640) + readout(x2_ref, 313, 320)
             + readout(x3_ref, 157, 160))
        h = jnp.maximum(jnp.dot(r, w1_ref[...],
                                preferred_element_type=jnp.float32)
                        + b1_ref[0:1, :], 0.0)
        h = jnp.maximum(jnp.dot(h, w2_ref[...],
                                preferred_element_type=jnp.float32)
                        + b2_ref[0:1, :64], 0.0)
        z = jnp.dot(h, w3_ref[...], preferred_element_type=jnp.float32) \
            + b3_ref[0:1, :10]
        m = jnp.max(z, axis=1, keepdims=True)
        e = jnp.exp(z - m)
        out_ref[...] = (z - m) - jnp.log(jnp.sum(e, axis=1, keepdims=True))

    return pl.pallas_call(
        body,
        out_shape=jax.ShapeDtypeStruct((G, 10), jnp.float32),
    )(x1, x2, x3, lin1_W, lin1_b, lin2_W, lin2_b, lin3_W, lin3_b)


def _blockdiag(W):
    H, di, do = W.shape
    M = jnp.zeros((H * di, H * do), jnp.float32)
    for i in range(H):
        M = M.at[i * di:(i + 1) * di, i * do:(i + 1) * do].set(W[i])
    return M


def _pad128(v):
    return jnp.pad(jnp.broadcast_to(v.reshape(1, -1), (8, v.shape[-1])),
                   ((0, 0), (0, 128 - v.shape[-1])))


def kernel(x, edge_index, batch,
           conv_W1, conv_b1, att_W1, att_b1, score_W1, score_b1,
           conv_W2, conv_b2, att_W2, att_b2, score_W2, score_b2,
           conv_W3, conv_b3, att_W3, att_b3, score_W3, score_b3,
           lin1_W, lin1_b, lin2_W, lin2_b, lin3_W, lin3_b):
    src = jnp.pad(edge_index[0].astype(jnp.int32), (0, EP - E0),
                  constant_values=-1)
    dst = jnp.pad(edge_index[1].astype(jnp.int32), (0, EP - E0),
                  constant_values=-1)
    pck = jnp.concatenate([src, dst])

    convs = ((conv_W1, conv_b1, score_W1, score_b1),
             (conv_W2, conv_b2, score_W2, score_b2),
             (conv_W3, conv_b3, score_W3, score_b3))

    xcur = jnp.pad(x, ((0, STAGES[0]["Np"] - N0), (0, 0)))
    ni_prev = None
    readouts = []

    for si, cfg in enumerate(STAGES):
        Np, stride, n_real = cfg["Np"], cfg["stride"], cfg["n_real"]
        k, kp_next, L = cfg["k"], cfg["kp_next"], cfg["L"]
        cW, cb, sW, sb = convs[si]

        if si == 0:
            pass_a = _make_pass_a(Np, False, 0)
            (deg_p,) = pass_a(pck)
            cnts = None
            cnts_prev = None
        else:
            pass_a = _make_pass_a(Np, True, ni_prev.shape[0])
            if cnts_prev is None:
                cnts_prev = jnp.full((NW * 16,), EW, jnp.int32)
            pck, cnts, deg_p = pass_a(pck, ni_prev, cnts_prev)

        acc_rows = Np + TRASH
        d0c = deg_p[:Np].reshape(Np, 1)
        d1c = deg_p[acc_rows:acc_rows + Np].reshape(Np, 1)
        Wbd = _blockdiag(cW)
        xs, dinv_c = _tc_prep(xcur, Wbd, d0c, d1c, Np)

        pass_b = _make_pass_b(Np, dynamic=si > 0)
        if si == 0:
            acc_p, sdeg_p = pass_b(pck, xs, dinv_c.reshape(Np))
        else:
            acc_p, sdeg_p = pass_b(pck, cnts, xs, dinv_c.reshape(Np))

        b_cat = jnp.concatenate([cb[i] for i in range(4)])
        sparams = _pad128(jnp.stack([sW[0, 0], sb[0]]))
        sd0c = sdeg_p[:Np].reshape(Np, 1)
        sd1c = sdeg_p[acc_rows:acc_rows + Np].reshape(Np, 1)
        hs, score_c = _tc_post(acc_p, xs, dinv_c, sd0c, sd1c, _pad128(b_cat),
                               sparams, Np)

        sflat = score_c.reshape(Np)
        if si == 0:
            sreal = sflat[:N0].reshape(G, stride)
            sreal = jnp.pad(sreal, ((0, 0), (0, L - stride)),
                            constant_values=-3e38)
        else:
            sreal = sflat.reshape(G, L)
        ni3 = _tc_topk(sreal.reshape(G, 1, L), sreal.reshape(G, L, 1),
                       n_real, k, kp_next, L)

        if si == 0:
            ni_flat = jnp.pad(ni3[:, 0, :stride].reshape(-1),
                              (0, Np - N0), constant_values=-1)
        else:
            ni_flat = ni3[:, 0, :].reshape(-1)

        out_real = G * kp_next
        pass_c = _make_pass_c(Np, out_real)
        (x_new,) = pass_c(hs, ni_flat)

        readouts.append(x_new)
        ni_prev = ni_flat
        cnts_prev = cnts
        xcur = x_new[:out_real]

    out = _tc_final(readouts[0], readouts[1], readouts[2],
                    lin1_W, _pad128(lin1_b), lin2_W, _pad128(lin2_b),
                    lin3_W, _pad128(lin3_b))
    return out


# speedup vs baseline: 185.0801x; 1.1070x over previous
import functools
import math

import jax
import jax.numpy as jnp
from jax import lax
from jax.experimental import pallas as pl
from jax.experimental.pallas import tpu as pltpu
from jax.experimental.pallas import tpu_sc as plsc

N0 = 10000
E0 = 320000
D = 128
G = 8
NW = 32
EP = 327680
EW = EP // NW
EB = 80
NBE = EW // EB
TRASH = 128

STAGES = (
    dict(Np=10240, stride=1250, n_real=1250, k=625, kp_next=640, L=1280),
    dict(Np=5120, stride=640, n_real=625, k=313, kp_next=320, L=640),
    dict(Np=2560, stride=320, n_real=313, k=157, kp_next=160, L=320),
)

_MESH = dict(
    mesh=plsc.VectorSubcoreMesh(core_axis_name="c", subcore_axis_name="s"),
    compiler_params=pltpu.CompilerParams(
        use_tc_tiling_on_sc=False, needs_layout_passes=False),
)


def _worker_ids():
    c = lax.axis_index("c")
    s = lax.axis_index("s")
    return c, s, s * 2 + c


def _zero_rows(zbuf, shared, row0, nrows):
    nfull, rem = nrows // 16, nrows % 16

    def zrow(r, carry):
        pltpu.sync_copy(zbuf, shared.at[pl.ds(row0 + r * 16, 16), :])
        return carry

    lax.fori_loop(0, nfull, zrow, 0)
    if rem:
        pltpu.sync_copy(zbuf.at[pl.ds(0, rem), :],
                        shared.at[pl.ds(row0 + nfull * 16, rem), :])


def _zero_flat(zbuf, shared, off, n):
    nfull, rem = n // 128, n % 128
    for r in range(nfull):
        pltpu.sync_copy(zbuf.at[0, :], shared.at[pl.ds(off + r * 128, 128)])
    if rem:
        pltpu.sync_copy(zbuf.at[0, pl.ds(0, rem)],
                        shared.at[pl.ds(off + nfull * 128, rem)])


def _fill_zbuf(zbuf):
    zeros = jnp.zeros((16,), jnp.float32)

    def body(r, carry):
        for c in range(8):
            zbuf[r, pl.ds(c * 16, 16)] = zeros
        return carry

    lax.fori_loop(0, 16, body, 0)


def _make_pass_a(Np, remap, ni_len):
    acc_rows = Np + TRASH
    epw = acc_rows // 16
    marg = EW + 2 * EB + 16

    out_type = [jax.ShapeDtypeStruct((2 * acc_rows,), jnp.float32)]
    if remap:
        out_type = [jax.ShapeDtypeStruct((2 * EP,), jnp.int32),
                    jax.ShapeDtypeStruct((NW * 16,), jnp.int32)] + out_type

    scratch = [
        pltpu.VMEM((EW,), jnp.int32),
        pltpu.VMEM((EW,), jnp.int32),
        pltpu.VMEM((1, EB), jnp.int32),
        pltpu.VMEM((1, EB), jnp.float32),
        pltpu.VMEM((16, 128), jnp.float32),
        pltpu.VMEM((epw,), jnp.float32),
    ]
    if remap:
        scratch = [pltpu.VMEM((ni_len,), jnp.int32),
                   pltpu.VMEM((marg,), jnp.int32),
                   pltpu.VMEM((marg,), jnp.int32),
                   pltpu.VMEM((16,), jnp.int32),
                   pltpu.VMEM((16,), jnp.int32)] + scratch

    shared = [pltpu.VMEM_SHARED((acc_rows,), jnp.float32)]

    @functools.partial(pl.kernel, out_type=out_type,
                       scratch_types=scratch + shared, **_MESH)
    def pass_a(*refs):
        if remap:
            (pck_h, ni_h, cntin_h, pcko_h, cnt_h, degp_h,
             ni_v, srcc_v, dstc_v, cnt_v, cntin_v,
             src_v, dst_v, idx_v, val_v, zbuf, bounce_v, deg_sh) = refs
        else:
            (pck_h, degp_h,
             src_v, dst_v, idx_v, val_v, zbuf, bounce_v, deg_sh) = refs

        c, s, w = _worker_ids()
        base_e = w * EW
        pltpu.sync_copy(pck_h.at[pl.ds(base_e, EW)], src_v)
        pltpu.sync_copy(pck_h.at[pl.ds(EP + base_e, EW)], dst_v)
        if remap:
            pltpu.sync_copy(ni_h, ni_v)
            pltpu.sync_copy(cntin_h.at[pl.ds(w * 16, 16)], cntin_v)
        _fill_zbuf(zbuf)
        _zero_flat(zbuf, deg_sh, s * epw, epw)
        plsc.subcore_barrier()

        lane = lax.iota(jnp.int32, 16)
        ones = jnp.full((16,), 1.0, jnp.float32)
        zerosf = jnp.zeros((16,), jnp.float32)

        def blk(b, cnt):
            for ch in range(EB // 16):
                o = b * EB + ch * 16
                sv = src_v[pl.ds(o, 16)]
                dv = dst_v[pl.ds(o, 16)]
                if remap:
                    sv = jnp.where(sv >= 0,
                                   plsc.load_gather(ni_v, [jnp.maximum(sv, 0)]),
                                   -1)
                    dv = jnp.where(dv >= 0,
                                   plsc.load_gather(ni_v, [jnp.maximum(dv, 0)]),
                                   -1)
                valid = (sv >= 0) & (dv >= 0)
                if remap:
                    plsc.store_compressed(srcc_v.at[pl.ds(cnt, 16)], sv,
                                          mask=valid)
                    plsc.store_compressed(dstc_v.at[pl.ds(cnt, 16)], dv,
                                          mask=valid)
                    cnt = cnt + jnp.sum(valid.astype(jnp.int32))
                spread = Np + ((b * EB + ch * 16 + lane) & (TRASH - 1))
                idx_v[0, pl.ds(ch * 16, 16)] = jnp.where(valid, dv, spread)
                val_v[0, pl.ds(ch * 16, 16)] = jnp.where(valid, ones, zerosf)
            pltpu.sync_copy(val_v.at[0], deg_sh.at[idx_v.at[0]], add=True)
            return cnt

        if remap:
            civ = cntin_v[pl.ds(0, 16)]
            nblk = (civ[0] + (EB - 1)) // EB
        else:
            nblk = NBE
        cnt = lax.fori_loop(0, nblk, blk, jnp.int32(0))

        if remap:
            neg = jnp.full((16,), -1, jnp.int32)
            for f in range(2 * EB // 16):
                srcc_v[pl.ds(cnt + f * 16, 16)] = neg
                dstc_v[pl.ds(cnt + f * 16, 16)] = neg
            pltpu.sync_copy(srcc_v.at[pl.ds(0, EW)],
                            pcko_h.at[pl.ds(base_e, EW)])
            pltpu.sync_copy(dstc_v.at[pl.ds(0, EW)],
                            pcko_h.at[pl.ds(EP + base_e, EW)])
            cnt_v[pl.ds(0, 16)] = jnp.full((16,), 1, jnp.int32) * cnt
            pltpu.sync_copy(cnt_v, cnt_h.at[pl.ds(w * 16, 16)])
        plsc.subcore_barrier()
        pltpu.sync_copy(deg_sh.at[pl.ds(s * epw, epw)], bounce_v)
        pltpu.sync_copy(bounce_v,
                        degp_h.at[pl.ds(c * acc_rows + s * epw, epw)])

    return pass_a


def _make_pass_b(Np, dynamic):
    acc_rows = Np + TRASH
    rpw = acc_rows // 16
    gmask = 2047

    @functools.partial(
        pl.kernel,
        out_type=[jax.ShapeDtypeStruct((2, acc_rows, D), jnp.float32),
                  jax.ShapeDtypeStruct((2 * acc_rows,), jnp.float32)],
        scratch_types=[
            pltpu.VMEM((EW,), jnp.int32),
            pltpu.VMEM((EW,), jnp.int32),
            pltpu.VMEM((2, EB), jnp.int32),
            pltpu.VMEM((2, EB), jnp.int32),
            pltpu.VMEM((2, EB), jnp.float32),
            pltpu.VMEM((2, EB, D), jnp.float32),
            pltpu.VMEM((16,), jnp.int32),
            pltpu.VMEM((16, 128), jnp.float32),
            pltpu.VMEM((rpw,), jnp.float32),
            pltpu.SemaphoreType.DMA,
            pltpu.SemaphoreType.DMA,
            pltpu.VMEM_SHARED((acc_rows, D), jnp.float32),
            pltpu.VMEM_SHARED((acc_rows,), jnp.float32),
        ],
        **_MESH)
    def pass_b(*refs):
        if dynamic:
            (pck_h, cnt_h, xs_h, dinv_h, accp_h, sdegp_h,
             src_v, dst_v, gidx_v, didx_v, sval_v, rows_v, cnt_v,
             zbuf, bounce_v, sem0, sem1, acc_sh, sdeg_sh) = refs
        else:
            (pck_h, xs_h, dinv_h, accp_h, sdegp_h,
             src_v, dst_v, gidx_v, didx_v, sval_v, rows_v, cnt_v,
             zbuf, bounce_v, sem0, sem1, acc_sh, sdeg_sh) = refs
        c, s, w = _worker_ids()
        base_e = w * EW
        pltpu.sync_copy(pck_h.at[pl.ds(base_e, EW)], src_v)
        pltpu.sync_copy(pck_h.at[pl.ds(EP + base_e, EW)], dst_v)
        if dynamic:
            pltpu.sync_copy(cnt_h.at[pl.ds(w * 16, 16)], cnt_v)
        _fill_zbuf(zbuf)
        _zero_rows(zbuf, acc_sh, s * rpw, rpw)
        _zero_flat(zbuf, sdeg_sh, s * rpw, rpw)
        plsc.subcore_barrier()

        lane = lax.iota(jnp.int32, 16)
        sems = (sem0, sem1)

        def blk(i, carry):
            handles = []
            for p in range(2):
                b = 2 * i + p
                for ch in range(EB // 16):
                    o = b * EB + ch * 16
                    sv = src_v[pl.ds(o, 16)]
                    dv = dst_v[pl.ds(o, 16)]
                    valid = (sv >= 0) & (dv >= 0)
                    pos = o + lane
                    gidx_v[p, pl.ds(ch * 16, 16)] = jnp.where(
                        valid, jnp.maximum(sv, 0), pos & gmask)
                    didx_v[p, pl.ds(ch * 16, 16)] = jnp.where(
                        valid, dv, Np + (pos & (TRASH - 1)))
                handles.append(
                    pltpu.async_copy(xs_h.at[gidx_v.at[p]], rows_v.at[p],
                                     sems[p]))
                handles.append(
                    pltpu.async_copy(dinv_h.at[gidx_v.at[p]], sval_v.at[p],
                                     sems[p]))
            for p in range(2):
                handles[2 * p].wait()
                handles[2 * p + 1].wait()
                pltpu.sync_copy(rows_v.at[p], acc_sh.at[didx_v.at[p]], add=True)
                pltpu.sync_copy(sval_v.at[p], sdeg_sh.at[didx_v.at[p]], add=True)
            return carry

        if dynamic:
            cv = cnt_v[pl.ds(0, 16)]
            npairs = (cv[0] + (2 * EB - 1)) // (2 * EB)
            lax.fori_loop(0, npairs, blk, 0)
        else:
            lax.fori_loop(0, NBE // 2, blk, 0)

        plsc.subcore_barrier()
        nfull, rem = rpw // EB, rpw % EB
        for r in range(nfull):
            r0 = s * rpw + r * EB
            pltpu.sync_copy(acc_sh.at[pl.ds(r0, EB), :], rows_v.at[r % 2])
            pltpu.sync_copy(rows_v.at[r % 2], accp_h.at[c, pl.ds(r0, EB), :])
        if rem:
            r0 = s * rpw + nfull * EB
            pltpu.sync_copy(acc_sh.at[pl.ds(r0, rem), :],
                            rows_v.at[0, pl.ds(0, rem), :])
            pltpu.sync_copy(rows_v.at[0, pl.ds(0, rem), :],
                            accp_h.at[c, pl.ds(r0, rem), :])
        pltpu.sync_copy(sdeg_sh.at[pl.ds(s * rpw, rpw)], bounce_v)
        pltpu.sync_copy(bounce_v,
                        sdegp_h.at[pl.ds(c * acc_rows + s * rpw, rpw)])

    return pass_b


def _make_pass_c(Np, out_real):
    rpw = Np // NW
    scb = 80
    nb = rpw // scb
    out_rows = out_real + TRASH

    @functools.partial(
        pl.kernel,
        out_type=[jax.ShapeDtypeStruct((out_rows, D), jnp.float32)],
        scratch_types=[
            pltpu.VMEM((rpw,), jnp.int32),
            pltpu.VMEM((rpw, D), jnp.float32),
            pltpu.VMEM((1, scb), jnp.int32),
        ],
        **_MESH)
    def pass_c(hs_h, ni_h, xnew_h, ni_v, rows_v, idx_v):
        c, s, w = _worker_ids()
        base = w * rpw
        pltpu.sync_copy(ni_h.at[pl.ds(base, rpw)], ni_v)
        pltpu.sync_copy(hs_h.at[pl.ds(base, rpw), :], rows_v)
        lane = lax.iota(jnp.int32, 16)
        for b in range(nb):
            for ch in range(scb // 16):
                o = b * scb + ch * 16
                ni = ni_v[pl.ds(o, 16)]
                spread = out_real + ((o + lane) & (TRASH - 1))
                idx_v[0, pl.ds(ch * 16, 16)] = jnp.where(ni >= 0, ni, spread)
            pltpu.sync_copy(rows_v.at[pl.ds(b * scb, scb), :],
                            xnew_h.at[idx_v.at[0]])

    return pass_c


_RB = 512


def _tc_prep(x, Wbd, d0c, d1c, Np):
    nb = Np // _RB

    def body(x_ref, w_ref, d0_ref, d1_ref, xs_ref, dinv_ref):
        deg = d0_ref[...] + d1_ref[...] + 1.0
        dinv = lax.rsqrt(deg)
        xw = jnp.dot(x_ref[...], w_ref[...], preferred_element_type=jnp.float32)
        xs_ref[...] = xw * dinv
        dinv_ref[...] = dinv

    return pl.pallas_call(
        body,
        grid=(nb,),
        in_specs=[
            pl.BlockSpec((_RB, D), lambda i: (i, 0)),
            pl.BlockSpec((D, D), lambda i: (0, 0)),
            pl.BlockSpec((_RB, 1), lambda i: (i, 0)),
            pl.BlockSpec((_RB, 1), lambda i: (i, 0)),
        ],
        out_specs=[
            pl.BlockSpec((_RB, D), lambda i: (i, 0)),
            pl.BlockSpec((_RB, 1), lambda i: (i, 0)),
        ],
        out_shape=[jax.ShapeDtypeStruct((Np, D), jnp.float32),
                   jax.ShapeDtypeStruct((Np, 1), jnp.float32)],
    )(x, Wbd, d0c, d1c)


def _tc_post(acc_p, xs, dinv_c, sd0c, sd1c, b_pad, sparams, Np):
    nb = Np // _RB

    def body(a0_ref, a1_ref, xs_ref, dinv_ref, sd0_ref, sd1_ref, b_ref,
             p_ref, hs_ref, score_ref):
        acc = a0_ref[0] + a1_ref[0] + xs_ref[...]
        h = jnp.maximum(dinv_ref[...] * acc + b_ref[0:1, :], 0.0)
        sd = sd0_ref[...] + sd1_ref[...]
        dinv = dinv_ref[...]
        sw = p_ref[0, 0]
        sb = p_ref[0, 1]
        score = sw * (dinv * sd + dinv * dinv) + sb
        hs_ref[...] = h * jnp.tanh(score)
        score_ref[...] = score

    return pl.pallas_call(
        body,
        grid=(nb,),
        in_specs=[
            pl.BlockSpec((1, _RB, D), lambda i: (0, i, 0)),
            pl.BlockSpec((1, _RB, D), lambda i: (1, i, 0)),
            pl.BlockSpec((_RB, D), lambda i: (i, 0)),
            pl.BlockSpec((_RB, 1), lambda i: (i, 0)),
            pl.BlockSpec((_RB, 1), lambda i: (i, 0)),
            pl.BlockSpec((_RB, 1), lambda i: (i, 0)),
            pl.BlockSpec((8, 128), lambda i: (0, 0)),
            pl.BlockSpec((8, 128), lambda i: (0, 0)),
        ],
        out_specs=[
            pl.BlockSpec((_RB, D), lambda i: (i, 0)),
            pl.BlockSpec((_RB, 1), lambda i: (i, 0)),
        ],
        out_shape=[jax.ShapeDtypeStruct((Np, D), jnp.float32),
                   jax.ShapeDtypeStruct((Np, 1), jnp.float32)],
    )(acc_p, acc_p, xs, dinv_c, sd0c, sd1c, b_pad, sparams)


def _tc_topk(s_row, s_col, n_real, k, kp_next, L):
    NEG = -3e38

    def body(sr_ref, sc_ref, ni_ref):
        g = pl.program_id(0)
        srow = sr_ref[0, 0, :]
        scol = sc_ref[0, :, :]
        li = lax.broadcasted_iota(jnp.int32, (L,), 0)
        real_r = li < n_real
        srow = jnp.where(real_r, srow, NEG)
        cj = lax.broadcasted_iota(jnp.int32, (L, 1), 0)
        real_c = cj < n_real
        scol = jnp.where(real_c, scol, NEG)
        jj = lax.broadcasted_iota(jnp.int32, (L, L), 0)
        ii = lax.broadcasted_iota(jnp.int32, (L, L), 1)
        beats = (scol > srow) | ((scol == srow) & (jj < ii))
        cnt = jnp.sum(beats.astype(jnp.float32), axis=0).astype(jnp.int32)
        sel = real_r & (cnt < k)
        ni_ref[0, 0, :] = jnp.where(sel, g * kp_next + cnt, -1)

    return pl.pallas_call(
        body,
        grid=(G,),
        in_specs=[
            pl.BlockSpec((1, 1, L), lambda g: (g, 0, 0)),
            pl.BlockSpec((1, L, 1), lambda g: (g, 0, 0)),
        ],
        out_specs=pl.BlockSpec((1, 1, L), lambda g: (g, 0, 0)),
        out_shape=jax.ShapeDtypeStruct((G, 1, L), jnp.int32),
    )(s_row, s_col)


def _tc_final(x1, x2, x3, lin1_W, lin1_b, lin2_W, lin2_b, lin3_W, lin3_b):
    def body(x1_ref, x2_ref, x3_ref, w1_ref, b1_ref, w2_ref, b2_ref,
             w3_ref, b3_ref, out_ref):
        def readout(ref, k, kp):
            mx, mn = [], []
            for g in range(G):
                xg = ref[g * kp:g * kp + k, :]
                mx.append(jnp.max(xg, axis=0, keepdims=True))
                mn.append(jnp.sum(xg, axis=0, keepdims=True) / k)
            return jnp.concatenate(
                [jnp.concatenate(mx, axis=0), jnp.concatenate(mn, axis=0)],
                axis=1)

        r = (readout(x1_ref, 625, 640) + readout(x2_ref, 313, 320)
             + readout(x3_ref, 157, 160))
        h = jnp.maximum(jnp.dot(r, w1_ref[...],
                                preferred_element_type=jnp.float32)
                        + b1_ref[0:1, :], 0.0)
        h = jnp.maximum(jnp.dot(h, w2_ref[...],
                                preferred_element_type=jnp.float32)
                        + b2_ref[0:1, :64], 0.0)
        z = jnp.dot(h, w3_ref[...], preferred_element_type=jnp.float32) \
            + b3_ref[0:1, :10]
        m = jnp.max(z, axis=1, keepdims=True)
        e = jnp.exp(z - m)
        out_ref[...] = (z - m) - jnp.log(jnp.sum(e, axis=1, keepdims=True))

    return pl.pallas_call(
        body,
        out_shape=jax.ShapeDtypeStruct((G, 10), jnp.float32),
    )(x1, x2, x3, lin1_W, lin1_b, lin2_W, lin2_b, lin3_W, lin3_b)


def _blockdiag(W):
    H, di, do = W.shape
    M = jnp.zeros((H * di, H * do), jnp.float32)
    for i in range(H):
        M = M.at[i * di:(i + 1) * di, i * do:(i + 1) * do].set(W[i])
    return M


def _pad128(v):
    return jnp.pad(jnp.broadcast_to(v.reshape(1, -1), (8, v.shape[-1])),
                   ((0, 0), (0, 128 - v.shape[-1])))


def kernel(x, edge_index, batch,
           conv_W1, conv_b1, att_W1, att_b1, score_W1, score_b1,
           conv_W2, conv_b2, att_W2, att_b2, score_W2, score_b2,
           conv_W3, conv_b3, att_W3, att_b3, score_W3, score_b3,
           lin1_W, lin1_b, lin2_W, lin2_b, lin3_W, lin3_b):
    src = jnp.pad(edge_index[0].astype(jnp.int32), (0, EP - E0),
                  constant_values=-1)
    dst = jnp.pad(edge_index[1].astype(jnp.int32), (0, EP - E0),
                  constant_values=-1)
    pck = jnp.concatenate([src, dst])

    convs = ((conv_W1, conv_b1, score_W1, score_b1),
             (conv_W2, conv_b2, score_W2, score_b2),
             (conv_W3, conv_b3, score_W3, score_b3))

    xcur = jnp.pad(x, ((0, STAGES[0]["Np"] - N0), (0, 0)))
    ni_prev = None
    readouts = []

    for si, cfg in enumerate(STAGES):
        Np, stride, n_real = cfg["Np"], cfg["stride"], cfg["n_real"]
        k, kp_next, L = cfg["k"], cfg["kp_next"], cfg["L"]
        cW, cb, sW, sb = convs[si]

        if si == 0:
            pass_a = _make_pass_a(Np, False, 0)
            (deg_p,) = pass_a(pck)
            cnts = None
            cnts_prev = None
        else:
            pass_a = _make_pass_a(Np, True, ni_prev.shape[0])
            if cnts_prev is None:
                cnts_prev = jnp.full((NW * 16,), EW, jnp.int32)
            pck, cnts, deg_p = pass_a(pck, ni_prev, cnts_prev)

        acc_rows = Np + TRASH
        d0c = deg_p[:Np].reshape(Np, 1)
        d1c = deg_p[acc_rows:acc_rows + Np].reshape(Np, 1)
        Wbd = _blockdiag(cW)
        xs, dinv_c = _tc_prep(xcur, Wbd, d0c, d1c, Np)

        pass_b = _make_pass_b(Np, dynamic=si > 0)
        if si == 0:
            acc_p, sdeg_p = pass_b(pck, xs, dinv_c.reshape(Np))
        else:
            acc_p, sdeg_p = pass_b(pck, cnts, xs, dinv_c.reshape(Np))

        b_cat = jnp.concatenate([cb[i] for i in range(4)])
        sparams = _pad128(jnp.stack([sW[0, 0], sb[0]]))
        sd0c = sdeg_p[:Np].reshape(Np, 1)
        sd1c = sdeg_p[acc_rows:acc_rows + Np].reshape(Np, 1)
        hs, score_c = _tc_post(acc_p, xs, dinv_c, sd0c, sd1c, _pad128(b_cat),
                               sparams, Np)

        sflat = score_c.reshape(Np)
        if si == 0:
            sreal = sflat[:N0].reshape(G, stride)
            sreal = jnp.pad(sreal, ((0, 0), (0, L - stride)),
                            constant_values=-3e38)
        else:
            sreal = sflat.reshape(G, L)
        ni3 = _tc_topk(sreal.reshape(G, 1, L), sreal.reshape(G, L, 1),
                       n_real, k, kp_next, L)

        if si == 0:
            ni_flat = jnp.pad(ni3[:, 0, :stride].reshape(-1),
                              (0, Np - N0), constant_values=-1)
        else:
            ni_flat = ni3[:, 0, :].reshape(-1)

        out_real = G * kp_next
        pass_c = _make_pass_c(Np, out_real)
        (x_new,) = pass_c(hs, ni_flat)

        readouts.append(x_new)
        ni_prev = ni_flat
        cnts_prev = cnts
        xcur = x_new[:out_real]

    out = _tc_final(readouts[0], readouts[1], readouts[2],
                    lin1_W, _pad128(lin1_b), lin2_W, _pad128(lin2_b),
                    lin3_W, _pad128(lin3_b))
    return out


# speedup vs baseline: 200.3010x; 1.0822x over previous
import functools
import math

import jax
import jax.numpy as jnp
from jax import lax
from jax.experimental import pallas as pl
from jax.experimental.pallas import tpu as pltpu
from jax.experimental.pallas import tpu_sc as plsc

N0 = 10000
E0 = 320000
D = 128
G = 8
NW = 32
EP = 327680
EW = EP // NW
EB = 80
NBE = EW // EB
TRASH = 128

STAGES = (
    dict(Np=10240, stride=1250, n_real=1250, k=625, kp_next=640, L=1280),
    dict(Np=5120, stride=640, n_real=625, k=313, kp_next=320, L=640),
    dict(Np=2560, stride=320, n_real=313, k=157, kp_next=160, L=320),
)

_MESH = dict(
    mesh=plsc.VectorSubcoreMesh(core_axis_name="c", subcore_axis_name="s"),
    compiler_params=pltpu.CompilerParams(
        use_tc_tiling_on_sc=False, needs_layout_passes=False),
)


def _worker_ids():
    c = lax.axis_index("c")
    s = lax.axis_index("s")
    return c, s, s * 2 + c


def _zero_rows(zbuf, shared, row0, nrows):
    nfull, rem = nrows // 16, nrows % 16

    def zrow(r, carry):
        pltpu.sync_copy(zbuf, shared.at[pl.ds(row0 + r * 16, 16), :])
        return carry

    lax.fori_loop(0, nfull, zrow, 0)
    if rem:
        pltpu.sync_copy(zbuf.at[pl.ds(0, rem), :],
                        shared.at[pl.ds(row0 + nfull * 16, rem), :])


def _zero_flat(zbuf, shared, off, n):
    nfull, rem = n // 128, n % 128
    for r in range(nfull):
        pltpu.sync_copy(zbuf.at[0, :], shared.at[pl.ds(off + r * 128, 128)])
    if rem:
        pltpu.sync_copy(zbuf.at[0, pl.ds(0, rem)],
                        shared.at[pl.ds(off + nfull * 128, rem)])


def _fill_zbuf(zbuf):
    zeros = jnp.zeros((16,), jnp.float32)

    def body(r, carry):
        for c in range(8):
            zbuf[r, pl.ds(c * 16, 16)] = zeros
        return carry

    lax.fori_loop(0, 16, body, 0)


def _make_pass_a(Np, remap, ni_len):
    acc_rows = Np + TRASH
    epw = acc_rows // 16
    marg = EW + 2 * EB + 16

    out_type = [jax.ShapeDtypeStruct((2 * acc_rows,), jnp.float32)]
    if remap:
        out_type = [jax.ShapeDtypeStruct((2 * EP,), jnp.int32),
                    jax.ShapeDtypeStruct((NW * 16,), jnp.int32)] + out_type

    scratch = [
        pltpu.VMEM((EW,), jnp.int32),
        pltpu.VMEM((EW,), jnp.int32),
        pltpu.VMEM((1, EB), jnp.int32),
        pltpu.VMEM((1, EB), jnp.float32),
        pltpu.VMEM((16, 128), jnp.float32),
        pltpu.VMEM((epw,), jnp.float32),
    ]
    if remap:
        scratch = [pltpu.VMEM((ni_len,), jnp.int32),
                   pltpu.VMEM((marg,), jnp.int32),
                   pltpu.VMEM((marg,), jnp.int32),
                   pltpu.VMEM((16,), jnp.int32),
                   pltpu.VMEM((16,), jnp.int32)] + scratch

    shared = [pltpu.VMEM_SHARED((acc_rows,), jnp.float32)]

    @functools.partial(pl.kernel, out_type=out_type,
                       scratch_types=scratch + shared, **_MESH)
    def pass_a(*refs):
        if remap:
            (pck_h, ni_h, cntin_h, pcko_h, cnt_h, degp_h,
             ni_v, srcc_v, dstc_v, cnt_v, cntin_v,
             src_v, dst_v, idx_v, val_v, zbuf, bounce_v, deg_sh) = refs
        else:
            (pck_h, degp_h,
             src_v, dst_v, idx_v, val_v, zbuf, bounce_v, deg_sh) = refs

        c, s, w = _worker_ids()
        base_e = w * EW
        pltpu.sync_copy(pck_h.at[pl.ds(base_e, EW)], src_v)
        pltpu.sync_copy(pck_h.at[pl.ds(EP + base_e, EW)], dst_v)
        if remap:
            pltpu.sync_copy(ni_h, ni_v)
            pltpu.sync_copy(cntin_h.at[pl.ds(w * 16, 16)], cntin_v)
        _fill_zbuf(zbuf)
        _zero_flat(zbuf, deg_sh, s * epw, epw)
        plsc.subcore_barrier()

        lane = lax.iota(jnp.int32, 16)
        ones = jnp.full((16,), 1.0, jnp.float32)
        zerosf = jnp.zeros((16,), jnp.float32)

        def blk(b, cnt):
            for ch in range(EB // 16):
                o = b * EB + ch * 16
                sv = src_v[pl.ds(o, 16)]
                dv = dst_v[pl.ds(o, 16)]
                if remap:
                    sv = jnp.where(sv >= 0,
                                   plsc.load_gather(ni_v, [jnp.maximum(sv, 0)]),
                                   -1)
                    dv = jnp.where(dv >= 0,
                                   plsc.load_gather(ni_v, [jnp.maximum(dv, 0)]),
                                   -1)
                valid = (sv >= 0) & (dv >= 0)
                if remap:
                    plsc.store_compressed(srcc_v.at[pl.ds(cnt, 16)], sv,
                                          mask=valid)
                    plsc.store_compressed(dstc_v.at[pl.ds(cnt, 16)], dv,
                                          mask=valid)
                    cnt = cnt + jnp.sum(valid.astype(jnp.int32))
                spread = Np + ((b * EB + ch * 16 + lane) & (TRASH - 1))
                idx_v[0, pl.ds(ch * 16, 16)] = jnp.where(valid, dv, spread)
                val_v[0, pl.ds(ch * 16, 16)] = jnp.where(valid, ones, zerosf)
            pltpu.sync_copy(val_v.at[0], deg_sh.at[idx_v.at[0]], add=True)
            return cnt

        if remap:
            civ = cntin_v[pl.ds(0, 16)]
            nblk = (civ[0] + (EB - 1)) // EB
        else:
            nblk = NBE
        cnt = lax.fori_loop(0, nblk, blk, jnp.int32(0))

        if remap:
            neg = jnp.full((16,), -1, jnp.int32)
            for f in range(2 * EB // 16):
                srcc_v[pl.ds(cnt + f * 16, 16)] = neg
                dstc_v[pl.ds(cnt + f * 16, 16)] = neg
            pltpu.sync_copy(srcc_v.at[pl.ds(0, EW)],
                            pcko_h.at[pl.ds(base_e, EW)])
            pltpu.sync_copy(dstc_v.at[pl.ds(0, EW)],
                            pcko_h.at[pl.ds(EP + base_e, EW)])
            cnt_v[pl.ds(0, 16)] = jnp.full((16,), 1, jnp.int32) * cnt
            pltpu.sync_copy(cnt_v, cnt_h.at[pl.ds(w * 16, 16)])
        plsc.subcore_barrier()
        pltpu.sync_copy(deg_sh.at[pl.ds(s * epw, epw)], bounce_v)
        pltpu.sync_copy(bounce_v,
                        degp_h.at[pl.ds(c * acc_rows + s * epw, epw)])

    return pass_a


def _make_pass_b(Np, dynamic):
    acc_rows = Np + TRASH
    rpw = acc_rows // 16
    gmask = 2047

    @functools.partial(
        pl.kernel,
        out_type=[jax.ShapeDtypeStruct((2, acc_rows, D), jnp.float32),
                  jax.ShapeDtypeStruct((2 * acc_rows,), jnp.float32)],
        scratch_types=[
            pltpu.VMEM((EW,), jnp.int32),
            pltpu.VMEM((EW,), jnp.int32),
            pltpu.VMEM((2, EB), jnp.int32),
            pltpu.VMEM((2, EB), jnp.int32),
            pltpu.VMEM((2, EB), jnp.float32),
            pltpu.VMEM((2, EB, D), jnp.float32),
            pltpu.VMEM((16,), jnp.int32),
            pltpu.VMEM((16, 128), jnp.float32),
            pltpu.VMEM((rpw,), jnp.float32),
            pltpu.SemaphoreType.DMA,
            pltpu.SemaphoreType.DMA,
            pltpu.SemaphoreType.DMA,
            pltpu.SemaphoreType.DMA,
            pltpu.SemaphoreType.DMA,
            pltpu.SemaphoreType.DMA,
            pltpu.VMEM_SHARED((acc_rows, D), jnp.float32),
            pltpu.VMEM_SHARED((acc_rows,), jnp.float32),
        ],
        **_MESH)
    def pass_b(*refs):
        if dynamic:
            (pck_h, cnt_h, xs_h, dinv_h, accp_h, sdegp_h,
             src_v, dst_v, gidx_v, didx_v, sval_v, rows_v, cnt_v,
             zbuf, bounce_v, sg0, sg1, ss0, ss1, sv0, sv1,
             acc_sh, sdeg_sh) = refs
        else:
            (pck_h, xs_h, dinv_h, accp_h, sdegp_h,
             src_v, dst_v, gidx_v, didx_v, sval_v, rows_v, cnt_v,
             zbuf, bounce_v, sg0, sg1, ss0, ss1, sv0, sv1,
             acc_sh, sdeg_sh) = refs
        c, s, w = _worker_ids()
        base_e = w * EW
        pltpu.sync_copy(pck_h.at[pl.ds(base_e, EW)], src_v)
        pltpu.sync_copy(pck_h.at[pl.ds(EP + base_e, EW)], dst_v)
        if dynamic:
            pltpu.sync_copy(cnt_h.at[pl.ds(w * 16, 16)], cnt_v)
        _fill_zbuf(zbuf)
        _zero_rows(zbuf, acc_sh, s * rpw, rpw)
        _zero_flat(zbuf, sdeg_sh, s * rpw, rpw)
        plsc.subcore_barrier()

        lane = lax.iota(jnp.int32, 16)
        semg = (sg0, sg1)
        sems = (ss0, ss1)
        semv = (sv0, sv1)

        for p in range(2):
            for ch in range(EB // 16):
                didx_v[p, pl.ds(ch * 16, 16)] = Np + ((ch * 16 + lane)
                                                      & (TRASH - 1))
            pltpu.async_copy(rows_v.at[p], acc_sh.at[didx_v.at[p]],
                             sems[p], add=True)
            pltpu.async_copy(sval_v.at[p], sdeg_sh.at[didx_v.at[p]],
                             semv[p], add=True)

        def _drain(p):
            pltpu.make_async_copy(xs_h.at[pl.ds(0, EB), :], rows_v.at[p],
                                  sems[p]).wait()
            pltpu.make_async_copy(dinv_h.at[pl.ds(0, EB)], sval_v.at[p],
                                  semv[p]).wait()

        def blk(i, carry):
            handles = []
            for p in range(2):
                b = 2 * i + p
                _drain(p)
                for ch in range(EB // 16):
                    o = b * EB + ch * 16
                    sv = src_v[pl.ds(o, 16)]
                    dv = dst_v[pl.ds(o, 16)]
                    valid = (sv >= 0) & (dv >= 0)
                    pos = o + lane
                    gidx_v[p, pl.ds(ch * 16, 16)] = jnp.where(
                        valid, jnp.maximum(sv, 0), pos & gmask)
                    didx_v[p, pl.ds(ch * 16, 16)] = jnp.where(
                        valid, dv, Np + (pos & (TRASH - 1)))
                handles.append(
                    pltpu.async_copy(xs_h.at[gidx_v.at[p]], rows_v.at[p],
                                     semg[p]))
                handles.append(
                    pltpu.async_copy(dinv_h.at[gidx_v.at[p]], sval_v.at[p],
                                     semg[p]))
            for p in range(2):
                handles[2 * p].wait()
                handles[2 * p + 1].wait()
                pltpu.async_copy(rows_v.at[p], acc_sh.at[didx_v.at[p]],
                                 sems[p], add=True)
                pltpu.async_copy(sval_v.at[p], sdeg_sh.at[didx_v.at[p]],
                                 semv[p], add=True)
            return carry

        if dynamic:
            cv = cnt_v[pl.ds(0, 16)]
            npairs = (cv[0] + (2 * EB - 1)) // (2 * EB)
            lax.fori_loop(0, npairs, blk, 0)
        else:
            lax.fori_loop(0, NBE // 2, blk, 0)
        for p in range(2):
            _drain(p)

        plsc.subcore_barrier()
        nfull, rem = rpw // EB, rpw % EB
        for r in range(nfull):
            r0 = s * rpw + r * EB
            pltpu.sync_copy(acc_sh.at[pl.ds(r0, EB), :], rows_v.at[r % 2])
            pltpu.sync_copy(rows_v.at[r % 2], accp_h.at[c, pl.ds(r0, EB), :])
        if rem:
            r0 = s * rpw + nfull * EB
            pltpu.sync_copy(acc_sh.at[pl.ds(r0, rem), :],
                            rows_v.at[0, pl.ds(0, rem), :])
            pltpu.sync_copy(rows_v.at[0, pl.ds(0, rem), :],
                            accp_h.at[c, pl.ds(r0, rem), :])
        pltpu.sync_copy(sdeg_sh.at[pl.ds(s * rpw, rpw)], bounce_v)
        pltpu.sync_copy(bounce_v,
                        sdegp_h.at[pl.ds(c * acc_rows + s * rpw, rpw)])

    return pass_b


def _make_pass_c(Np, out_real):
    rpw = Np // NW
    scb = 80
    nb = rpw // scb
    out_rows = out_real + TRASH

    @functools.partial(
        pl.kernel,
        out_type=[jax.ShapeDtypeStruct((out_rows, D), jnp.float32)],
        scratch_types=[
            pltpu.VMEM((rpw,), jnp.int32),
            pltpu.VMEM((rpw, D), jnp.float32),
            pltpu.VMEM((1, scb), jnp.int32),
        ],
        **_MESH)
    def pass_c(hs_h, ni_h, xnew_h, ni_v, rows_v, idx_v):
        c, s, w = _worker_ids()
        base = w * rpw
        pltpu.sync_copy(ni_h.at[pl.ds(base, rpw)], ni_v)
        pltpu.sync_copy(hs_h.at[pl.ds(base, rpw), :], rows_v)
        lane = lax.iota(jnp.int32, 16)
        for b in range(nb):
            for ch in range(scb // 16):
                o = b * scb + ch * 16
                ni = ni_v[pl.ds(o, 16)]
                spread = out_real + ((o + lane) & (TRASH - 1))
                idx_v[0, pl.ds(ch * 16, 16)] = jnp.where(ni >= 0, ni, spread)
            pltpu.sync_copy(rows_v.at[pl.ds(b * scb, scb), :],
                            xnew_h.at[idx_v.at[0]])

    return pass_c


_RB = 512


def _tc_prep(x, Wbd, d0c, d1c, Np):
    nb = Np // _RB

    def body(x_ref, w_ref, d0_ref, d1_ref, xs_ref, dinv_ref):
        deg = d0_ref[...] + d1_ref[...] + 1.0
        dinv = lax.rsqrt(deg)
        xw = jnp.dot(x_ref[...], w_ref[...], preferred_element_type=jnp.float32)
        xs_ref[...] = xw * dinv
        dinv_ref[...] = dinv

    return pl.pallas_call(
        body,
        grid=(nb,),
        in_specs=[
            pl.BlockSpec((_RB, D), lambda i: (i, 0)),
            pl.BlockSpec((D, D), lambda i: (0, 0)),
            pl.BlockSpec((_RB, 1), lambda i: (i, 0)),
            pl.BlockSpec((_RB, 1), lambda i: (i, 0)),
        ],
        out_specs=[
            pl.BlockSpec((_RB, D), lambda i: (i, 0)),
            pl.BlockSpec((_RB, 1), lambda i: (i, 0)),
        ],
        out_shape=[jax.ShapeDtypeStruct((Np, D), jnp.float32),
                   jax.ShapeDtypeStruct((Np, 1), jnp.float32)],
    )(x, Wbd, d0c, d1c)


def _tc_post(acc_p, xs, dinv_c, sd0c, sd1c, b_pad, sparams, Np):
    nb = Np // _RB

    def body(a0_ref, a1_ref, xs_ref, dinv_ref, sd0_ref, sd1_ref, b_ref,
             p_ref, hs_ref, score_ref):
        acc = a0_ref[0] + a1_ref[0] + xs_ref[...]
        h = jnp.maximum(dinv_ref[...] * acc + b_ref[0:1, :], 0.0)
        sd = sd0_ref[...] + sd1_ref[...]
        dinv = dinv_ref[...]
        sw = p_ref[0, 0]
        sb = p_ref[0, 1]
        score = sw * (dinv * sd + dinv * dinv) + sb
        hs_ref[...] = h * jnp.tanh(score)
        score_ref[...] = score

    return pl.pallas_call(
        body,
        grid=(nb,),
        in_specs=[
            pl.BlockSpec((1, _RB, D), lambda i: (0, i, 0)),
            pl.BlockSpec((1, _RB, D), lambda i: (1, i, 0)),
            pl.BlockSpec((_RB, D), lambda i: (i, 0)),
            pl.BlockSpec((_RB, 1), lambda i: (i, 0)),
            pl.BlockSpec((_RB, 1), lambda i: (i, 0)),
            pl.BlockSpec((_RB, 1), lambda i: (i, 0)),
            pl.BlockSpec((8, 128), lambda i: (0, 0)),
            pl.BlockSpec((8, 128), lambda i: (0, 0)),
        ],
        out_specs=[
            pl.BlockSpec((_RB, D), lambda i: (i, 0)),
            pl.BlockSpec((_RB, 1), lambda i: (i, 0)),
        ],
        out_shape=[jax.ShapeDtypeStruct((Np, D), jnp.float32),
                   jax.ShapeDtypeStruct((Np, 1), jnp.float32)],
    )(acc_p, acc_p, xs, dinv_c, sd0c, sd1c, b_pad, sparams)


def _tc_topk(s_row, s_col, n_real, k, kp_next, L):
    NEG = -3e38

    def body(sr_ref, sc_ref, ni_ref):
        g = pl.program_id(0)
        srow = sr_ref[0, 0, :]
        scol = sc_ref[0, :, :]
        li = lax.broadcasted_iota(jnp.int32, (L,), 0)
        real_r = li < n_real
        srow = jnp.where(real_r, srow, NEG)
        cj = lax.broadcasted_iota(jnp.int32, (L, 1), 0)
        real_c = cj < n_real
        scol = jnp.where(real_c, scol, NEG)
        jj = lax.broadcasted_iota(jnp.int32, (L, L), 0)
        ii = lax.broadcasted_iota(jnp.int32, (L, L), 1)
        beats = (scol > srow) | ((scol == srow) & (jj < ii))
        cnt = jnp.sum(beats.astype(jnp.float32), axis=0).astype(jnp.int32)
        sel = real_r & (cnt < k)
        ni_ref[0, 0, :] = jnp.where(sel, g * kp_next + cnt, -1)

    return pl.pallas_call(
        body,
        grid=(G,),
        in_specs=[
            pl.BlockSpec((1, 1, L), lambda g: (g, 0, 0)),
            pl.BlockSpec((1, L, 1), lambda g: (g, 0, 0)),
        ],
        out_specs=pl.BlockSpec((1, 1, L), lambda g: (g, 0, 0)),
        out_shape=jax.ShapeDtypeStruct((G, 1, L), jnp.int32),
    )(s_row, s_col)


def _tc_final(x1, x2, x3, lin1_W, lin1_b, lin2_W, lin2_b, lin3_W, lin3_b):
    def body(x1_ref, x2_ref, x3_ref, w1_ref, b1_ref, w2_ref, b2_ref,
             w3_ref, b3_ref, out_ref):
        def readout(ref, k, kp):
            mx, mn = [], []
            for g in range(G):
                xg = ref[g * kp:g * kp + k, :]
                mx.append(jnp.max(xg, axis=0, keepdims=True))
                mn.append(jnp.sum(xg, axis=0, keepdims=True) / k)
            return jnp.concatenate(
                [jnp.concatenate(mx, axis=0), jnp.concatenate(mn, axis=0)],
                axis=1)

        r = (readout(x1_ref, 625, 640) + readout(x2_ref, 313, 320)
             + readout(x3_ref, 157, 160))
        h = jnp.maximum(jnp.dot(r, w1_ref[...],
                                preferred_element_type=jnp.float32)
                        + b1_ref[0:1, :], 0.0)
        h = jnp.maximum(jnp.dot(h, w2_ref[...],
                                preferred_element_type=jnp.float32)
                        + b2_ref[0:1, :64], 0.0)
        z = jnp.dot(h, w3_ref[...], preferred_element_type=jnp.float32) \
            + b3_ref[0:1, :10]
        m = jnp.max(z, axis=1, keepdims=True)
        e = jnp.exp(z - m)
        out_ref[...] = (z - m) - jnp.log(jnp.sum(e, axis=1, keepdims=True))

    return pl.pallas_call(
        body,
        out_shape=jax.ShapeDtypeStruct((G, 10), jnp.float32),
    )(x1, x2, x3, lin1_W, lin1_b, lin2_W, lin2_b, lin3_W, lin3_b)


def _blockdiag(W):
    H, di, do = W.shape
    M = jnp.zeros((H * di, H * do), jnp.float32)
    for i in range(H):
        M = M.at[i * di:(i + 1) * di, i * do:(i + 1) * do].set(W[i])
    return M


def _pad128(v):
    return jnp.pad(jnp.broadcast_to(v.reshape(1, -1), (8, v.shape[-1])),
                   ((0, 0), (0, 128 - v.shape[-1])))


def kernel(x, edge_index, batch,
           conv_W1, conv_b1, att_W1, att_b1, score_W1, score_b1,
           conv_W2, conv_b2, att_W2, att_b2, score_W2, score_b2,
           conv_W3, conv_b3, att_W3, att_b3, score_W3, score_b3,
           lin1_W, lin1_b, lin2_W, lin2_b, lin3_W, lin3_b):
    src = jnp.pad(edge_index[0].astype(jnp.int32), (0, EP - E0),
                  constant_values=-1)
    dst = jnp.pad(edge_index[1].astype(jnp.int32), (0, EP - E0),
                  constant_values=-1)
    pck = jnp.concatenate([src, dst])

    convs = ((conv_W1, conv_b1, score_W1, score_b1),
             (conv_W2, conv_b2, score_W2, score_b2),
             (conv_W3, conv_b3, score_W3, score_b3))

    xcur = jnp.pad(x, ((0, STAGES[0]["Np"] - N0), (0, 0)))
    ni_prev = None
    readouts = []

    for si, cfg in enumerate(STAGES):
        Np, stride, n_real = cfg["Np"], cfg["stride"], cfg["n_real"]
        k, kp_next, L = cfg["k"], cfg["kp_next"], cfg["L"]
        cW, cb, sW, sb = convs[si]

        if si == 0:
            pass_a = _make_pass_a(Np, False, 0)
            (deg_p,) = pass_a(pck)
            cnts = None
            cnts_prev = None
        else:
            pass_a = _make_pass_a(Np, True, ni_prev.shape[0])
            if cnts_prev is None:
                cnts_prev = jnp.full((NW * 16,), EW, jnp.int32)
            pck, cnts, deg_p = pass_a(pck, ni_prev, cnts_prev)

        acc_rows = Np + TRASH
        d0c = deg_p[:Np].reshape(Np, 1)
        d1c = deg_p[acc_rows:acc_rows + Np].reshape(Np, 1)
        Wbd = _blockdiag(cW)
        xs, dinv_c = _tc_prep(xcur, Wbd, d0c, d1c, Np)

        pass_b = _make_pass_b(Np, dynamic=si > 0)
        if si == 0:
            acc_p, sdeg_p = pass_b(pck, xs, dinv_c.reshape(Np))
        else:
            acc_p, sdeg_p = pass_b(pck, cnts, xs, dinv_c.reshape(Np))

        b_cat = jnp.concatenate([cb[i] for i in range(4)])
        sparams = _pad128(jnp.stack([sW[0, 0], sb[0]]))
        sd0c = sdeg_p[:Np].reshape(Np, 1)
        sd1c = sdeg_p[acc_rows:acc_rows + Np].reshape(Np, 1)
        hs, score_c = _tc_post(acc_p, xs, dinv_c, sd0c, sd1c, _pad128(b_cat),
                               sparams, Np)

        sflat = score_c.reshape(Np)
        if si == 0:
            sreal = sflat[:N0].reshape(G, stride)
            sreal = jnp.pad(sreal, ((0, 0), (0, L - stride)),
                            constant_values=-3e38)
        else:
            sreal = sflat.reshape(G, L)
        ni3 = _tc_topk(sreal.reshape(G, 1, L), sreal.reshape(G, L, 1),
                       n_real, k, kp_next, L)

        if si == 0:
            ni_flat = jnp.pad(ni3[:, 0, :stride].reshape(-1),
                              (0, Np - N0), constant_values=-1)
        else:
            ni_flat = ni3[:, 0, :].reshape(-1)

        out_real = G * kp_next
        pass_c = _make_pass_c(Np, out_real)
        (x_new,) = pass_c(hs, ni_flat)

        readouts.append(x_new)
        ni_prev = ni_flat
        cnts_prev = cnts
        xcur = x_new[:out_real]

    out = _tc_final(readouts[0], readouts[1], readouts[2],
                    lin1_W, _pad128(lin1_b), lin2_W, _pad128(lin2_b),
                    lin3_W, _pad128(lin3_b))
    return out


# speedup vs baseline: 200.9691x; 1.0033x over previous
import functools
import math

import jax
import jax.numpy as jnp
from jax import lax
from jax.experimental import pallas as pl
from jax.experimental.pallas import tpu as pltpu
from jax.experimental.pallas import tpu_sc as plsc

N0 = 10000
E0 = 320000
D = 128
G = 8
NW = 32
EP = 327680
EW = EP // NW
EB = 80
NBE = EW // EB
TRASH = 128

STAGES = (
    dict(Np=10240, stride=1250, n_real=1250, k=625, kp_next=640, L=1280),
    dict(Np=5120, stride=640, n_real=625, k=313, kp_next=320, L=640),
    dict(Np=2560, stride=320, n_real=313, k=157, kp_next=160, L=320),
)

_MESH = dict(
    mesh=plsc.VectorSubcoreMesh(core_axis_name="c", subcore_axis_name="s"),
    compiler_params=pltpu.CompilerParams(
        use_tc_tiling_on_sc=False, needs_layout_passes=False),
)


def _worker_ids():
    c = lax.axis_index("c")
    s = lax.axis_index("s")
    return c, s, s * 2 + c


def _zero_rows(zbuf, shared, row0, nrows):
    nfull, rem = nrows // 16, nrows % 16

    def zrow(r, carry):
        pltpu.sync_copy(zbuf, shared.at[pl.ds(row0 + r * 16, 16), :])
        return carry

    lax.fori_loop(0, nfull, zrow, 0)
    if rem:
        pltpu.sync_copy(zbuf.at[pl.ds(0, rem), :],
                        shared.at[pl.ds(row0 + nfull * 16, rem), :])


def _zero_flat(zbuf, shared, off, n):
    nfull, rem = n // 128, n % 128
    for r in range(nfull):
        pltpu.sync_copy(zbuf.at[0, :], shared.at[pl.ds(off + r * 128, 128)])
    if rem:
        pltpu.sync_copy(zbuf.at[0, pl.ds(0, rem)],
                        shared.at[pl.ds(off + nfull * 128, rem)])


def _fill_zbuf(zbuf):
    zeros = jnp.zeros((16,), jnp.float32)

    def body(r, carry):
        for c in range(8):
            zbuf[r, pl.ds(c * 16, 16)] = zeros
        return carry

    lax.fori_loop(0, 16, body, 0)


def _make_pass_a(Np, remap, ni_len):
    acc_rows = Np + TRASH
    epw = acc_rows // 16
    marg = EW + 2 * EB + 16
    prev_np = ni_len
    rpw_c = prev_np // NW if remap else 0
    scb = 80

    out_type = [jax.ShapeDtypeStruct((2 * acc_rows,), jnp.float32)]
    if remap:
        out_type = [jax.ShapeDtypeStruct((2 * EP,), jnp.int32),
                    jax.ShapeDtypeStruct((NW * 16,), jnp.int32),
                    jax.ShapeDtypeStruct((Np + TRASH, D), jnp.float32)] \
            + out_type

    scratch = [
        pltpu.VMEM((EW,), jnp.int32),
        pltpu.VMEM((EW,), jnp.int32),
        pltpu.VMEM((1, EB), jnp.int32),
        pltpu.VMEM((1, EB), jnp.float32),
        pltpu.VMEM((16, 128), jnp.float32),
        pltpu.VMEM((epw,), jnp.float32),
    ]
    if remap:
        scratch = [pltpu.VMEM((ni_len,), jnp.int32),
                   pltpu.VMEM((marg,), jnp.int32),
                   pltpu.VMEM((marg,), jnp.int32),
                   pltpu.VMEM((16,), jnp.int32),
                   pltpu.VMEM((16,), jnp.int32),
                   pltpu.VMEM((rpw_c, D), jnp.float32),
                   pltpu.VMEM((1, scb), jnp.int32)] + scratch

    shared = [pltpu.VMEM_SHARED((acc_rows,), jnp.float32)]

    @functools.partial(pl.kernel, out_type=out_type,
                       scratch_types=scratch + shared, **_MESH)
    def pass_a(*refs):
        if remap:
            (pck_h, ni_h, cntin_h, hs_h, pcko_h, cnt_h, xnew_h, degp_h,
             ni_v, srcc_v, dstc_v, cnt_v, cntin_v, rows_v, cidx_v,
             src_v, dst_v, idx_v, val_v, zbuf, bounce_v, deg_sh) = refs
        else:
            (pck_h, degp_h,
             src_v, dst_v, idx_v, val_v, zbuf, bounce_v, deg_sh) = refs

        c, s, w = _worker_ids()
        base_e = w * EW
        pltpu.sync_copy(pck_h.at[pl.ds(base_e, EW)], src_v)
        pltpu.sync_copy(pck_h.at[pl.ds(EP + base_e, EW)], dst_v)
        if remap:
            pltpu.sync_copy(ni_h, ni_v)
            pltpu.sync_copy(cntin_h.at[pl.ds(w * 16, 16)], cntin_v)
        _fill_zbuf(zbuf)
        _zero_flat(zbuf, deg_sh, s * epw, epw)

        if remap:
            lane0 = lax.iota(jnp.int32, 16)
            base_r = w * rpw_c
            pltpu.sync_copy(hs_h.at[pl.ds(base_r, rpw_c), :], rows_v)
            for b in range(rpw_c // scb):
                for ch in range(scb // 16):
                    o = b * scb + ch * 16
                    ni = ni_v[pl.ds(base_r + o, 16)]
                    spread = Np + ((o + lane0) & (TRASH - 1))
                    cidx_v[0, pl.ds(ch * 16, 16)] = jnp.where(ni >= 0, ni,
                                                              spread)
                pltpu.sync_copy(rows_v.at[pl.ds(b * scb, scb), :],
                                xnew_h.at[cidx_v.at[0]])
        plsc.subcore_barrier()

        lane = lax.iota(jnp.int32, 16)
        ones = jnp.full((16,), 1.0, jnp.float32)
        zerosf = jnp.zeros((16,), jnp.float32)

        def blk(b, cnt):
            for ch in range(EB // 16):
                o = b * EB + ch * 16
                sv = src_v[pl.ds(o, 16)]
                dv = dst_v[pl.ds(o, 16)]
                if remap:
                    sv = jnp.where(sv >= 0,
                                   plsc.load_gather(ni_v, [jnp.maximum(sv, 0)]),
                                   -1)
                    dv = jnp.where(dv >= 0,
                                   plsc.load_gather(ni_v, [jnp.maximum(dv, 0)]),
                                   -1)
                valid = (sv >= 0) & (dv >= 0)
                if remap:
                    plsc.store_compressed(srcc_v.at[pl.ds(cnt, 16)], sv,
                                          mask=valid)
                    plsc.store_compressed(dstc_v.at[pl.ds(cnt, 16)], dv,
                                          mask=valid)
                    cnt = cnt + jnp.sum(valid.astype(jnp.int32))
                spread = Np + ((b * EB + ch * 16 + lane) & (TRASH - 1))
                idx_v[0, pl.ds(ch * 16, 16)] = jnp.where(valid, dv, spread)
                val_v[0, pl.ds(ch * 16, 16)] = jnp.where(valid, ones, zerosf)
            pltpu.sync_copy(val_v.at[0], deg_sh.at[idx_v.at[0]], add=True)
            return cnt

        if remap:
            civ = cntin_v[pl.ds(0, 16)]
            nblk = (civ[0] + (EB - 1)) // EB
        else:
            nblk = NBE
        cnt = lax.fori_loop(0, nblk, blk, jnp.int32(0))

        if remap:
            neg = jnp.full((16,), -1, jnp.int32)
            for f in range(2 * EB // 16):
                srcc_v[pl.ds(cnt + f * 16, 16)] = neg
                dstc_v[pl.ds(cnt + f * 16, 16)] = neg
            pltpu.sync_copy(srcc_v.at[pl.ds(0, EW)],
                            pcko_h.at[pl.ds(base_e, EW)])
            pltpu.sync_copy(dstc_v.at[pl.ds(0, EW)],
                            pcko_h.at[pl.ds(EP + base_e, EW)])
            cnt_v[pl.ds(0, 16)] = jnp.full((16,), 1, jnp.int32) * cnt
            pltpu.sync_copy(cnt_v, cnt_h.at[pl.ds(w * 16, 16)])
        plsc.subcore_barrier()
        pltpu.sync_copy(deg_sh.at[pl.ds(s * epw, epw)], bounce_v)
        pltpu.sync_copy(bounce_v,
                        degp_h.at[pl.ds(c * acc_rows + s * epw, epw)])

    return pass_a


def _make_pass_b(Np, dynamic):
    acc_rows = Np + TRASH
    rpw = acc_rows // 16
    gmask = 2047

    @functools.partial(
        pl.kernel,
        out_type=[jax.ShapeDtypeStruct((2, acc_rows, D), jnp.float32),
                  jax.ShapeDtypeStruct((2 * acc_rows,), jnp.float32)],
        scratch_types=[
            pltpu.VMEM((EW,), jnp.int32),
            pltpu.VMEM((EW,), jnp.int32),
            pltpu.VMEM((2, EB), jnp.int32),
            pltpu.VMEM((2, EB), jnp.int32),
            pltpu.VMEM((2, EB), jnp.float32),
            pltpu.VMEM((2, EB, D), jnp.float32),
            pltpu.VMEM((16,), jnp.int32),
            pltpu.VMEM((16, 128), jnp.float32),
            pltpu.VMEM((rpw,), jnp.float32),
            pltpu.SemaphoreType.DMA,
            pltpu.SemaphoreType.DMA,
            pltpu.SemaphoreType.DMA,
            pltpu.SemaphoreType.DMA,
            pltpu.SemaphoreType.DMA,
            pltpu.SemaphoreType.DMA,
            pltpu.VMEM_SHARED((acc_rows, D), jnp.float32),
            pltpu.VMEM_SHARED((acc_rows,), jnp.float32),
        ],
        **_MESH)
    def pass_b(*refs):
        if dynamic:
            (pck_h, cnt_h, xs_h, dinv_h, accp_h, sdegp_h,
             src_v, dst_v, gidx_v, didx_v, sval_v, rows_v, cnt_v,
             zbuf, bounce_v, sg0, sg1, ss0, ss1, sv0, sv1,
             acc_sh, sdeg_sh) = refs
        else:
            (pck_h, xs_h, dinv_h, accp_h, sdegp_h,
             src_v, dst_v, gidx_v, didx_v, sval_v, rows_v, cnt_v,
             zbuf, bounce_v, sg0, sg1, ss0, ss1, sv0, sv1,
             acc_sh, sdeg_sh) = refs
        c, s, w = _worker_ids()
        base_e = w * EW
        pltpu.sync_copy(pck_h.at[pl.ds(base_e, EW)], src_v)
        pltpu.sync_copy(pck_h.at[pl.ds(EP + base_e, EW)], dst_v)
        if dynamic:
            pltpu.sync_copy(cnt_h.at[pl.ds(w * 16, 16)], cnt_v)
        _fill_zbuf(zbuf)
        _zero_rows(zbuf, acc_sh, s * rpw, rpw)
        _zero_flat(zbuf, sdeg_sh, s * rpw, rpw)
        plsc.subcore_barrier()

        lane = lax.iota(jnp.int32, 16)
        semg = (sg0, sg1)
        sems = (ss0, ss1)
        semv = (sv0, sv1)

        for p in range(2):
            for ch in range(EB // 16):
                didx_v[p, pl.ds(ch * 16, 16)] = Np + ((ch * 16 + lane)
                                                      & (TRASH - 1))
            pltpu.async_copy(rows_v.at[p], acc_sh.at[didx_v.at[p]],
                             sems[p], add=True)
            pltpu.async_copy(sval_v.at[p], sdeg_sh.at[didx_v.at[p]],
                             semv[p], add=True)

        def _drain(p):
            pltpu.make_async_copy(xs_h.at[pl.ds(0, EB), :], rows_v.at[p],
                                  sems[p]).wait()
            pltpu.make_async_copy(dinv_h.at[pl.ds(0, EB)], sval_v.at[p],
                                  semv[p]).wait()

        def blk(i, carry):
            handles = []
            for p in range(2):
                b = 2 * i + p
                _drain(p)
                for ch in range(EB // 16):
                    o = b * EB + ch * 16
                    sv = src_v[pl.ds(o, 16)]
                    dv = dst_v[pl.ds(o, 16)]
                    valid = (sv >= 0) & (dv >= 0)
                    pos = o + lane
                    gidx_v[p, pl.ds(ch * 16, 16)] = jnp.where(
                        valid, jnp.maximum(sv, 0), pos & gmask)
                    didx_v[p, pl.ds(ch * 16, 16)] = jnp.where(
                        valid, dv, Np + (pos & (TRASH - 1)))
                handles.append(
                    pltpu.async_copy(xs_h.at[gidx_v.at[p]], rows_v.at[p],
                                     semg[p]))
                handles.append(
                    pltpu.async_copy(dinv_h.at[gidx_v.at[p]], sval_v.at[p],
                                     semg[p]))
            for p in range(2):
                handles[2 * p].wait()
                handles[2 * p + 1].wait()
                pltpu.async_copy(rows_v.at[p], acc_sh.at[didx_v.at[p]],
                                 sems[p], add=True)
                pltpu.async_copy(sval_v.at[p], sdeg_sh.at[didx_v.at[p]],
                                 semv[p], add=True)
            return carry

        if dynamic:
            cv = cnt_v[pl.ds(0, 16)]
            npairs = (cv[0] + (2 * EB - 1)) // (2 * EB)
            lax.fori_loop(0, npairs, blk, 0)
        else:
            lax.fori_loop(0, NBE // 2, blk, 0)
        for p in range(2):
            _drain(p)

        plsc.subcore_barrier()
        nfull, rem = rpw // EB, rpw % EB
        for r in range(nfull):
            r0 = s * rpw + r * EB
            pltpu.sync_copy(acc_sh.at[pl.ds(r0, EB), :], rows_v.at[r % 2])
            pltpu.sync_copy(rows_v.at[r % 2], accp_h.at[c, pl.ds(r0, EB), :])
        if rem:
            r0 = s * rpw + nfull * EB
            pltpu.sync_copy(acc_sh.at[pl.ds(r0, rem), :],
                            rows_v.at[0, pl.ds(0, rem), :])
            pltpu.sync_copy(rows_v.at[0, pl.ds(0, rem), :],
                            accp_h.at[c, pl.ds(r0, rem), :])
        pltpu.sync_copy(sdeg_sh.at[pl.ds(s * rpw, rpw)], bounce_v)
        pltpu.sync_copy(bounce_v,
                        sdegp_h.at[pl.ds(c * acc_rows + s * rpw, rpw)])

    return pass_b


def _make_pass_c(Np, out_real):
    rpw = Np // NW
    scb = 80
    nb = rpw // scb
    out_rows = out_real + TRASH

    @functools.partial(
        pl.kernel,
        out_type=[jax.ShapeDtypeStruct((out_rows, D), jnp.float32)],
        scratch_types=[
            pltpu.VMEM((rpw,), jnp.int32),
            pltpu.VMEM((rpw, D), jnp.float32),
            pltpu.VMEM((1, scb), jnp.int32),
        ],
        **_MESH)
    def pass_c(hs_h, ni_h, xnew_h, ni_v, rows_v, idx_v):
        c, s, w = _worker_ids()
        base = w * rpw
        pltpu.sync_copy(ni_h.at[pl.ds(base, rpw)], ni_v)
        pltpu.sync_copy(hs_h.at[pl.ds(base, rpw), :], rows_v)
        lane = lax.iota(jnp.int32, 16)
        for b in range(nb):
            for ch in range(scb // 16):
                o = b * scb + ch * 16
                ni = ni_v[pl.ds(o, 16)]
                spread = out_real + ((o + lane) & (TRASH - 1))
                idx_v[0, pl.ds(ch * 16, 16)] = jnp.where(ni >= 0, ni, spread)
            pltpu.sync_copy(rows_v.at[pl.ds(b * scb, scb), :],
                            xnew_h.at[idx_v.at[0]])

    return pass_c


_RB = 512


def _tc_prep(x, Wbd, d0c, d1c, Np):
    nb = Np // _RB

    def body(x_ref, w_ref, d0_ref, d1_ref, xs_ref, dinv_ref):
        deg = d0_ref[...] + d1_ref[...] + 1.0
        dinv = lax.rsqrt(deg)
        xw = jnp.dot(x_ref[...], w_ref[...], preferred_element_type=jnp.float32)
        xs_ref[...] = xw * dinv
        dinv_ref[...] = dinv

    return pl.pallas_call(
        body,
        grid=(nb,),
        in_specs=[
            pl.BlockSpec((_RB, D), lambda i: (i, 0)),
            pl.BlockSpec((D, D), lambda i: (0, 0)),
            pl.BlockSpec((_RB, 1), lambda i: (i, 0)),
            pl.BlockSpec((_RB, 1), lambda i: (i, 0)),
        ],
        out_specs=[
            pl.BlockSpec((_RB, D), lambda i: (i, 0)),
            pl.BlockSpec((_RB, 1), lambda i: (i, 0)),
        ],
        out_shape=[jax.ShapeDtypeStruct((Np, D), jnp.float32),
                   jax.ShapeDtypeStruct((Np, 1), jnp.float32)],
    )(x, Wbd, d0c, d1c)


def _tc_post(acc_p, xs, dinv_c, sd0c, sd1c, b_pad, sparams, Np):
    nb = Np // _RB

    def body(a0_ref, a1_ref, xs_ref, dinv_ref, sd0_ref, sd1_ref, b_ref,
             p_ref, hs_ref, score_ref):
        acc = a0_ref[0] + a1_ref[0] + xs_ref[...]
        h = jnp.maximum(dinv_ref[...] * acc + b_ref[0:1, :], 0.0)
        sd = sd0_ref[...] + sd1_ref[...]
        dinv = dinv_ref[...]
        sw = p_ref[0, 0]
        sb = p_ref[0, 1]
        score = sw * (dinv * sd + dinv * dinv) + sb
        hs_ref[...] = h * jnp.tanh(score)
        score_ref[...] = score

    return pl.pallas_call(
        body,
        grid=(nb,),
        in_specs=[
            pl.BlockSpec((1, _RB, D), lambda i: (0, i, 0)),
            pl.BlockSpec((1, _RB, D), lambda i: (1, i, 0)),
            pl.BlockSpec((_RB, D), lambda i: (i, 0)),
            pl.BlockSpec((_RB, 1), lambda i: (i, 0)),
            pl.BlockSpec((_RB, 1), lambda i: (i, 0)),
            pl.BlockSpec((_RB, 1), lambda i: (i, 0)),
            pl.BlockSpec((8, 128), lambda i: (0, 0)),
            pl.BlockSpec((8, 128), lambda i: (0, 0)),
        ],
        out_specs=[
            pl.BlockSpec((_RB, D), lambda i: (i, 0)),
            pl.BlockSpec((_RB, 1), lambda i: (i, 0)),
        ],
        out_shape=[jax.ShapeDtypeStruct((Np, D), jnp.float32),
                   jax.ShapeDtypeStruct((Np, 1), jnp.float32)],
    )(acc_p, acc_p, xs, dinv_c, sd0c, sd1c, b_pad, sparams)


def _tc_topk(s_row, s_col, n_real, k, kp_next, L):
    NEG = -3e38

    def body(sr_ref, sc_ref, ni_ref):
        g = pl.program_id(0)
        srow = sr_ref[0, 0, :]
        scol = sc_ref[0, :, :]
        li = lax.broadcasted_iota(jnp.int32, (L,), 0)
        real_r = li < n_real
        srow = jnp.where(real_r, srow, NEG)
        cj = lax.broadcasted_iota(jnp.int32, (L, 1), 0)
        real_c = cj < n_real
        scol = jnp.where(real_c, scol, NEG)
        jj = lax.broadcasted_iota(jnp.int32, (L, L), 0)
        ii = lax.broadcasted_iota(jnp.int32, (L, L), 1)
        beats = (scol > srow) | ((scol == srow) & (jj < ii))
        cnt = jnp.sum(beats.astype(jnp.float32), axis=0).astype(jnp.int32)
        sel = real_r & (cnt < k)
        ni_ref[0, 0, :] = jnp.where(sel, g * kp_next + cnt, -1)

    return pl.pallas_call(
        body,
        grid=(G,),
        in_specs=[
            pl.BlockSpec((1, 1, L), lambda g: (g, 0, 0)),
            pl.BlockSpec((1, L, 1), lambda g: (g, 0, 0)),
        ],
        out_specs=pl.BlockSpec((1, 1, L), lambda g: (g, 0, 0)),
        out_shape=jax.ShapeDtypeStruct((G, 1, L), jnp.int32),
    )(s_row, s_col)


def _tc_final(x1, x2, x3, lin1_W, lin1_b, lin2_W, lin2_b, lin3_W, lin3_b):
    def body(x1_ref, x2_ref, x3_ref, w1_ref, b1_ref, w2_ref, b2_ref,
             w3_ref, b3_ref, out_ref):
        def readout(ref, k, kp):
            mx, mn = [], []
            for g in range(G):
                xg = ref[g * kp:g * kp + k, :]
                mx.append(jnp.max(xg, axis=0, keepdims=True))
                mn.append(jnp.sum(xg, axis=0, keepdims=True) / k)
            return jnp.concatenate(
                [jnp.concatenate(mx, axis=0), jnp.concatenate(mn, axis=0)],
                axis=1)

        r = (readout(x1_ref, 625, 640) + readout(x2_ref, 313, 320)
             + readout(x3_ref, 157, 160))
        h = jnp.maximum(jnp.dot(r, w1_ref[...],
                                preferred_element_type=jnp.float32)
                        + b1_ref[0:1, :], 0.0)
        h = jnp.maximum(jnp.dot(h, w2_ref[...],
                                preferred_element_type=jnp.float32)
                        + b2_ref[0:1, :64], 0.0)
        z = jnp.dot(h, w3_ref[...], preferred_element_type=jnp.float32) \
            + b3_ref[0:1, :10]
        m = jnp.max(z, axis=1, keepdims=True)
        e = jnp.exp(z - m)
        out_ref[...] = (z - m) - jnp.log(jnp.sum(e, axis=1, keepdims=True))

    return pl.pallas_call(
        body,
        out_shape=jax.ShapeDtypeStruct((G, 10), jnp.float32),
    )(x1, x2, x3, lin1_W, lin1_b, lin2_W, lin2_b, lin3_W, lin3_b)


def _blockdiag(W):
    H, di, do = W.shape
    M = jnp.zeros((H * di, H * do), jnp.float32)
    for i in range(H):
        M = M.at[i * di:(i + 1) * di, i * do:(i + 1) * do].set(W[i])
    return M


def _pad128(v):
    return jnp.pad(jnp.broadcast_to(v.reshape(1, -1), (8, v.shape[-1])),
                   ((0, 0), (0, 128 - v.shape[-1])))


def kernel(x, edge_index, batch,
           conv_W1, conv_b1, att_W1, att_b1, score_W1, score_b1,
           conv_W2, conv_b2, att_W2, att_b2, score_W2, score_b2,
           conv_W3, conv_b3, att_W3, att_b3, score_W3, score_b3,
           lin1_W, lin1_b, lin2_W, lin2_b, lin3_W, lin3_b):
    src = jnp.pad(edge_index[0].astype(jnp.int32), (0, EP - E0),
                  constant_values=-1)
    dst = jnp.pad(edge_index[1].astype(jnp.int32), (0, EP - E0),
                  constant_values=-1)
    pck = jnp.concatenate([src, dst])

    convs = ((conv_W1, conv_b1, score_W1, score_b1),
             (conv_W2, conv_b2, score_W2, score_b2),
             (conv_W3, conv_b3, score_W3, score_b3))

    xcur = jnp.pad(x, ((0, STAGES[0]["Np"] - N0), (0, 0)))
    ni_prev = None
    readouts = []

    for si, cfg in enumerate(STAGES):
        Np, stride, n_real = cfg["Np"], cfg["stride"], cfg["n_real"]
        k, kp_next, L = cfg["k"], cfg["kp_next"], cfg["L"]
        cW, cb, sW, sb = convs[si]

        if si == 0:
            pass_a = _make_pass_a(Np, False, 0)
            (deg_p,) = pass_a(pck)
            cnts = None
            cnts_prev = None
        else:
            pass_a = _make_pass_a(Np, True, ni_prev.shape[0])
            if cnts_prev is None:
                cnts_prev = jnp.full((NW * 16,), EW, jnp.int32)
            pck, cnts, x_new_prev, deg_p = pass_a(pck, ni_prev, cnts_prev,
                                                  hs_prev)
            readouts.append(x_new_prev)
            xcur = x_new_prev[:Np]

        acc_rows = Np + TRASH
        d0c = deg_p[:Np].reshape(Np, 1)
        d1c = deg_p[acc_rows:acc_rows + Np].reshape(Np, 1)
        Wbd = _blockdiag(cW)
        xs, dinv_c = _tc_prep(xcur, Wbd, d0c, d1c, Np)

        pass_b = _make_pass_b(Np, dynamic=si > 0)
        if si == 0:
            acc_p, sdeg_p = pass_b(pck, xs, dinv_c.reshape(Np))
        else:
            acc_p, sdeg_p = pass_b(pck, cnts, xs, dinv_c.reshape(Np))

        b_cat = jnp.concatenate([cb[i] for i in range(4)])
        sparams = _pad128(jnp.stack([sW[0, 0], sb[0]]))
        sd0c = sdeg_p[:Np].reshape(Np, 1)
        sd1c = sdeg_p[acc_rows:acc_rows + Np].reshape(Np, 1)
        hs, score_c = _tc_post(acc_p, xs, dinv_c, sd0c, sd1c, _pad128(b_cat),
                               sparams, Np)

        sflat = score_c.reshape(Np)
        if si == 0:
            sreal = sflat[:N0].reshape(G, stride)
            sreal = jnp.pad(sreal, ((0, 0), (0, L - stride)),
                            constant_values=-3e38)
        else:
            sreal = sflat.reshape(G, L)
        ni3 = _tc_topk(sreal.reshape(G, 1, L), sreal.reshape(G, L, 1),
                       n_real, k, kp_next, L)

        if si == 0:
            ni_flat = jnp.pad(ni3[:, 0, :stride].reshape(-1),
                              (0, Np - N0), constant_values=-1)
        else:
            ni_flat = ni3[:, 0, :].reshape(-1)

        hs_prev = hs
        ni_prev = ni_flat
        cnts_prev = cnts

    pass_c = _make_pass_c(STAGES[2]["Np"], G * STAGES[2]["kp_next"])
    (x_new3,) = pass_c(hs_prev, ni_prev)
    readouts.append(x_new3)

    out = _tc_final(readouts[0], readouts[1], readouts[2],
                    lin1_W, _pad128(lin1_b), lin2_W, _pad128(lin2_b),
                    lin3_W, _pad128(lin3_b))
    return out


# speedup vs baseline: 203.3784x; 1.0120x over previous
import functools
import math

import jax
import jax.numpy as jnp
from jax import lax
from jax.experimental import pallas as pl
from jax.experimental.pallas import tpu as pltpu
from jax.experimental.pallas import tpu_sc as plsc

N0 = 10000
E0 = 320000
D = 128
G = 8
NW = 32
EP = 327680
EW = EP // NW
EB = 80
NBE = EW // EB
TRASH = 128

STAGES = (
    dict(Np=10240, stride=1250, n_real=1250, k=625, kp_next=640, L=1280),
    dict(Np=5120, stride=640, n_real=625, k=313, kp_next=320, L=640),
    dict(Np=2560, stride=320, n_real=313, k=157, kp_next=160, L=320),
)

_MESH = dict(
    mesh=plsc.VectorSubcoreMesh(core_axis_name="c", subcore_axis_name="s"),
    compiler_params=pltpu.CompilerParams(
        use_tc_tiling_on_sc=False, needs_layout_passes=False),
)


def _worker_ids():
    c = lax.axis_index("c")
    s = lax.axis_index("s")
    return c, s, s * 2 + c


def _zero_rows(zbuf, shared, row0, nrows, sem=None):
    nfull, rem = nrows // 16, nrows % 16
    if sem is not None:
        def zrow(r, carry):
            pltpu.async_copy(zbuf, shared.at[pl.ds(row0 + r * 16, 16), :], sem)
            return carry

        lax.fori_loop(0, nfull, zrow, 0)

        def zdrain(r, carry):
            pltpu.make_async_copy(zbuf, shared.at[pl.ds(row0, 16), :],
                                  sem).wait()
            return carry

        lax.fori_loop(0, nfull, zdrain, 0)
    else:
        def zrow(r, carry):
            pltpu.sync_copy(zbuf, shared.at[pl.ds(row0 + r * 16, 16), :])
            return carry

        lax.fori_loop(0, nfull, zrow, 0)
    if rem:
        pltpu.sync_copy(zbuf.at[pl.ds(0, rem), :],
                        shared.at[pl.ds(row0 + nfull * 16, rem), :])


def _zero_flat(zbuf, shared, off, n):
    nfull, rem = n // 128, n % 128
    for r in range(nfull):
        pltpu.sync_copy(zbuf.at[0, :], shared.at[pl.ds(off + r * 128, 128)])
    if rem:
        pltpu.sync_copy(zbuf.at[0, pl.ds(0, rem)],
                        shared.at[pl.ds(off + nfull * 128, rem)])


def _fill_zbuf(zbuf):
    zeros = jnp.zeros((16,), jnp.float32)

    def body(r, carry):
        for c in range(8):
            zbuf[r, pl.ds(c * 16, 16)] = zeros
        return carry

    lax.fori_loop(0, 16, body, 0)


def _make_pass_a(Np, remap, ni_len):
    acc_rows = Np + TRASH
    epw = acc_rows // 16
    marg = EW + 2 * EB + 16
    prev_np = ni_len
    rpw_c = prev_np // NW if remap else 0
    scb = 80

    out_type = [jax.ShapeDtypeStruct((2 * acc_rows,), jnp.float32)]
    if remap:
        out_type = [jax.ShapeDtypeStruct((2 * EP,), jnp.int32),
                    jax.ShapeDtypeStruct((NW * 16,), jnp.int32),
                    jax.ShapeDtypeStruct((Np + TRASH, D), jnp.float32)] \
            + out_type

    scratch = [
        pltpu.VMEM((EW,), jnp.int32),
        pltpu.VMEM((EW,), jnp.int32),
        pltpu.VMEM((1, EB), jnp.int32),
        pltpu.VMEM((1, EB), jnp.float32),
        pltpu.VMEM((16, 128), jnp.float32),
        pltpu.VMEM((epw,), jnp.float32),
    ]
    if remap:
        scratch = [pltpu.VMEM((ni_len,), jnp.int32),
                   pltpu.VMEM((marg,), jnp.int32),
                   pltpu.VMEM((marg,), jnp.int32),
                   pltpu.VMEM((16,), jnp.int32),
                   pltpu.VMEM((16,), jnp.int32),
                   pltpu.VMEM((rpw_c, D), jnp.float32),
                   pltpu.VMEM((1, scb), jnp.int32)] + scratch

    shared = [pltpu.VMEM_SHARED((acc_rows,), jnp.float32)]

    @functools.partial(pl.kernel, out_type=out_type,
                       scratch_types=scratch + shared, **_MESH)
    def pass_a(*refs):
        if remap:
            (pck_h, ni_h, cntin_h, hs_h, pcko_h, cnt_h, xnew_h, degp_h,
             ni_v, srcc_v, dstc_v, cnt_v, cntin_v, rows_v, cidx_v,
             src_v, dst_v, idx_v, val_v, zbuf, bounce_v, deg_sh) = refs
        else:
            (pck_h, degp_h,
             src_v, dst_v, idx_v, val_v, zbuf, bounce_v, deg_sh) = refs

        c, s, w = _worker_ids()
        base_e = w * EW
        pltpu.sync_copy(pck_h.at[pl.ds(base_e, EW)], src_v)
        pltpu.sync_copy(pck_h.at[pl.ds(EP + base_e, EW)], dst_v)
        if remap:
            pltpu.sync_copy(ni_h, ni_v)
            pltpu.sync_copy(cntin_h.at[pl.ds(w * 16, 16)], cntin_v)
        _fill_zbuf(zbuf)
        _zero_flat(zbuf, deg_sh, s * epw, epw)

        if remap:
            lane0 = lax.iota(jnp.int32, 16)
            base_r = w * rpw_c
            pltpu.sync_copy(hs_h.at[pl.ds(base_r, rpw_c), :], rows_v)
            for b in range(rpw_c // scb):
                for ch in range(scb // 16):
                    o = b * scb + ch * 16
                    ni = ni_v[pl.ds(base_r + o, 16)]
                    spread = Np + ((o + lane0) & (TRASH - 1))
                    cidx_v[0, pl.ds(ch * 16, 16)] = jnp.where(ni >= 0, ni,
                                                              spread)
                pltpu.sync_copy(rows_v.at[pl.ds(b * scb, scb), :],
                                xnew_h.at[cidx_v.at[0]])
        plsc.subcore_barrier()

        lane = lax.iota(jnp.int32, 16)
        ones = jnp.full((16,), 1.0, jnp.float32)
        zerosf = jnp.zeros((16,), jnp.float32)

        def blk(b, cnt):
            for ch in range(EB // 16):
                o = b * EB + ch * 16
                sv = src_v[pl.ds(o, 16)]
                dv = dst_v[pl.ds(o, 16)]
                if remap:
                    sv = jnp.where(sv >= 0,
                                   plsc.load_gather(ni_v, [jnp.maximum(sv, 0)]),
                                   -1)
                    dv = jnp.where(dv >= 0,
                                   plsc.load_gather(ni_v, [jnp.maximum(dv, 0)]),
                                   -1)
                valid = (sv >= 0) & (dv >= 0)
                if remap:
                    plsc.store_compressed(srcc_v.at[pl.ds(cnt, 16)], sv,
                                          mask=valid)
                    plsc.store_compressed(dstc_v.at[pl.ds(cnt, 16)], dv,
                                          mask=valid)
                    cnt = cnt + jnp.sum(valid.astype(jnp.int32))
                spread = Np + ((b * EB + ch * 16 + lane) & (TRASH - 1))
                idx_v[0, pl.ds(ch * 16, 16)] = jnp.where(valid, dv, spread)
                val_v[0, pl.ds(ch * 16, 16)] = jnp.where(valid, ones, zerosf)
            pltpu.sync_copy(val_v.at[0], deg_sh.at[idx_v.at[0]], add=True)
            return cnt

        if remap:
            civ = cntin_v[pl.ds(0, 16)]
            nblk = (civ[0] + (EB - 1)) // EB
        else:
            nblk = NBE
        cnt = lax.fori_loop(0, nblk, blk, jnp.int32(0))

        if remap:
            neg = jnp.full((16,), -1, jnp.int32)
            for f in range(2 * EB // 16):
                srcc_v[pl.ds(cnt + f * 16, 16)] = neg
                dstc_v[pl.ds(cnt + f * 16, 16)] = neg
            pltpu.sync_copy(srcc_v.at[pl.ds(0, EW)],
                            pcko_h.at[pl.ds(base_e, EW)])
            pltpu.sync_copy(dstc_v.at[pl.ds(0, EW)],
                            pcko_h.at[pl.ds(EP + base_e, EW)])
            cnt_v[pl.ds(0, 16)] = jnp.full((16,), 1, jnp.int32) * cnt
            pltpu.sync_copy(cnt_v, cnt_h.at[pl.ds(w * 16, 16)])
        plsc.subcore_barrier()
        pltpu.sync_copy(deg_sh.at[pl.ds(s * epw, epw)], bounce_v)
        pltpu.sync_copy(bounce_v,
                        degp_h.at[pl.ds(c * acc_rows + s * epw, epw)])

    return pass_a


def _make_pass_b(Np, dynamic):
    acc_rows = Np + TRASH
    rpw = acc_rows // 16
    gmask = 2047

    @functools.partial(
        pl.kernel,
        out_type=[jax.ShapeDtypeStruct((2, acc_rows, D), jnp.float32),
                  jax.ShapeDtypeStruct((2 * acc_rows,), jnp.float32)],
        scratch_types=[
            pltpu.VMEM((EW,), jnp.int32),
            pltpu.VMEM((EW,), jnp.int32),
            pltpu.VMEM((2, EB), jnp.int32),
            pltpu.VMEM((2, EB), jnp.int32),
            pltpu.VMEM((2, EB), jnp.float32),
            pltpu.VMEM((2, EB, D), jnp.float32),
            pltpu.VMEM((16,), jnp.int32),
            pltpu.VMEM((16, 128), jnp.float32),
            pltpu.VMEM((rpw,), jnp.float32),
            pltpu.SemaphoreType.DMA,
            pltpu.SemaphoreType.DMA,
            pltpu.SemaphoreType.DMA,
            pltpu.SemaphoreType.DMA,
            pltpu.SemaphoreType.DMA,
            pltpu.SemaphoreType.DMA,
            pltpu.VMEM_SHARED((acc_rows, D), jnp.float32),
            pltpu.VMEM_SHARED((acc_rows,), jnp.float32),
        ],
        **_MESH)
    def pass_b(*refs):
        if dynamic:
            (pck_h, cnt_h, xs_h, dinv_h, accp_h, sdegp_h,
             src_v, dst_v, gidx_v, didx_v, sval_v, rows_v, cnt_v,
             zbuf, bounce_v, sg0, sg1, ss0, ss1, sv0, sv1,
             acc_sh, sdeg_sh) = refs
        else:
            (pck_h, xs_h, dinv_h, accp_h, sdegp_h,
             src_v, dst_v, gidx_v, didx_v, sval_v, rows_v, cnt_v,
             zbuf, bounce_v, sg0, sg1, ss0, ss1, sv0, sv1,
             acc_sh, sdeg_sh) = refs
        c, s, w = _worker_ids()
        base_e = w * EW
        pltpu.sync_copy(pck_h.at[pl.ds(base_e, EW)], src_v)
        pltpu.sync_copy(pck_h.at[pl.ds(EP + base_e, EW)], dst_v)
        if dynamic:
            pltpu.sync_copy(cnt_h.at[pl.ds(w * 16, 16)], cnt_v)
        _fill_zbuf(zbuf)
        _zero_rows(zbuf, acc_sh, s * rpw, rpw, sem=sg0)
        _zero_flat(zbuf, sdeg_sh, s * rpw, rpw)
        plsc.subcore_barrier()

        lane = lax.iota(jnp.int32, 16)
        semg = (sg0, sg1)
        sems = (ss0, ss1)
        semv = (sv0, sv1)

        for p in range(2):
            for ch in range(EB // 16):
                didx_v[p, pl.ds(ch * 16, 16)] = Np + ((ch * 16 + lane)
                                                      & (TRASH - 1))
            pltpu.async_copy(rows_v.at[p], acc_sh.at[didx_v.at[p]],
                             sems[p], add=True)
            pltpu.async_copy(sval_v.at[p], sdeg_sh.at[didx_v.at[p]],
                             semv[p], add=True)

        def _drain(p):
            pltpu.make_async_copy(xs_h.at[pl.ds(0, EB), :], rows_v.at[p],
                                  sems[p]).wait()
            pltpu.make_async_copy(dinv_h.at[pl.ds(0, EB)], sval_v.at[p],
                                  semv[p]).wait()

        def blk(i, carry):
            handles = []
            for p in range(2):
                b = 2 * i + p
                _drain(p)
                for ch in range(EB // 16):
                    o = b * EB + ch * 16
                    sv = src_v[pl.ds(o, 16)]
                    dv = dst_v[pl.ds(o, 16)]
                    valid = (sv >= 0) & (dv >= 0)
                    pos = o + lane
                    gidx_v[p, pl.ds(ch * 16, 16)] = jnp.where(
                        valid, jnp.maximum(sv, 0), pos & gmask)
                    didx_v[p, pl.ds(ch * 16, 16)] = jnp.where(
                        valid, dv, Np + (pos & (TRASH - 1)))
                handles.append(
                    pltpu.async_copy(xs_h.at[gidx_v.at[p]], rows_v.at[p],
                                     semg[p]))
                handles.append(
                    pltpu.async_copy(dinv_h.at[gidx_v.at[p]], sval_v.at[p],
                                     semg[p]))
            for p in range(2):
                handles[2 * p].wait()
                handles[2 * p + 1].wait()
                pltpu.async_copy(rows_v.at[p], acc_sh.at[didx_v.at[p]],
                                 sems[p], add=True)
                pltpu.async_copy(sval_v.at[p], sdeg_sh.at[didx_v.at[p]],
                                 semv[p], add=True)
            return carry

        if dynamic:
            cv = cnt_v[pl.ds(0, 16)]
            npairs = (cv[0] + (2 * EB - 1)) // (2 * EB)
            lax.fori_loop(0, npairs, blk, 0)
        else:
            lax.fori_loop(0, NBE // 2, blk, 0)
        for p in range(2):
            _drain(p)

        plsc.subcore_barrier()
        nfull, rem = rpw // EB, rpw % EB
        semw = (sg0, sg1)
        pend = [None, None]
        for r in range(nfull):
            p = r % 2
            if pend[p] is not None:
                pend[p].wait()
            r0 = s * rpw + r * EB
            pltpu.sync_copy(acc_sh.at[pl.ds(r0, EB), :], rows_v.at[p])
            pend[p] = pltpu.async_copy(rows_v.at[p],
                                       accp_h.at[c, pl.ds(r0, EB), :],
                                       semw[p])
        for p in range(2):
            if pend[p] is not None:
                pend[p].wait()
        if rem:
            r0 = s * rpw + nfull * EB
            pltpu.sync_copy(acc_sh.at[pl.ds(r0, rem), :],
                            rows_v.at[0, pl.ds(0, rem), :])
            pltpu.sync_copy(rows_v.at[0, pl.ds(0, rem), :],
                            accp_h.at[c, pl.ds(r0, rem), :])
        pltpu.sync_copy(sdeg_sh.at[pl.ds(s * rpw, rpw)], bounce_v)
        pltpu.sync_copy(bounce_v,
                        sdegp_h.at[pl.ds(c * acc_rows + s * rpw, rpw)])

    return pass_b


def _make_pass_c(Np, out_real):
    rpw = Np // NW
    scb = 80
    nb = rpw // scb
    out_rows = out_real + TRASH

    @functools.partial(
        pl.kernel,
        out_type=[jax.ShapeDtypeStruct((out_rows, D), jnp.float32)],
        scratch_types=[
            pltpu.VMEM((rpw,), jnp.int32),
            pltpu.VMEM((rpw, D), jnp.float32),
            pltpu.VMEM((1, scb), jnp.int32),
        ],
        **_MESH)
    def pass_c(hs_h, ni_h, xnew_h, ni_v, rows_v, idx_v):
        c, s, w = _worker_ids()
        base = w * rpw
        pltpu.sync_copy(ni_h.at[pl.ds(base, rpw)], ni_v)
        pltpu.sync_copy(hs_h.at[pl.ds(base, rpw), :], rows_v)
        lane = lax.iota(jnp.int32, 16)
        for b in range(nb):
            for ch in range(scb // 16):
                o = b * scb + ch * 16
                ni = ni_v[pl.ds(o, 16)]
                spread = out_real + ((o + lane) & (TRASH - 1))
                idx_v[0, pl.ds(ch * 16, 16)] = jnp.where(ni >= 0, ni, spread)
            pltpu.sync_copy(rows_v.at[pl.ds(b * scb, scb), :],
                            xnew_h.at[idx_v.at[0]])

    return pass_c


_RB = 512


def _tc_prep(x, Wbd, d0c, d1c, Np):
    nb = Np // _RB

    def body(x_ref, w_ref, d0_ref, d1_ref, xs_ref, dinv_ref):
        deg = d0_ref[...] + d1_ref[...] + 1.0
        dinv = lax.rsqrt(deg)
        xw = jnp.dot(x_ref[...], w_ref[...], preferred_element_type=jnp.float32)
        xs_ref[...] = xw * dinv
        dinv_ref[...] = dinv

    return pl.pallas_call(
        body,
        grid=(nb,),
        in_specs=[
            pl.BlockSpec((_RB, D), lambda i: (i, 0)),
            pl.BlockSpec((D, D), lambda i: (0, 0)),
            pl.BlockSpec((_RB, 1), lambda i: (i, 0)),
            pl.BlockSpec((_RB, 1), lambda i: (i, 0)),
        ],
        out_specs=[
            pl.BlockSpec((_RB, D), lambda i: (i, 0)),
            pl.BlockSpec((_RB, 1), lambda i: (i, 0)),
        ],
        out_shape=[jax.ShapeDtypeStruct((Np, D), jnp.float32),
                   jax.ShapeDtypeStruct((Np, 1), jnp.float32)],
    )(x, Wbd, d0c, d1c)


def _tc_post(acc_p, xs, dinv_c, sd0c, sd1c, b_pad, sparams, Np):
    nb = Np // _RB

    def body(a0_ref, a1_ref, xs_ref, dinv_ref, sd0_ref, sd1_ref, b_ref,
             p_ref, hs_ref, score_ref):
        acc = a0_ref[0] + a1_ref[0] + xs_ref[...]
        h = jnp.maximum(dinv_ref[...] * acc + b_ref[0:1, :], 0.0)
        sd = sd0_ref[...] + sd1_ref[...]
        dinv = dinv_ref[...]
        sw = p_ref[0, 0]
        sb = p_ref[0, 1]
        score = sw * (dinv * sd + dinv * dinv) + sb
        hs_ref[...] = h * jnp.tanh(score)
        score_ref[...] = score

    return pl.pallas_call(
        body,
        grid=(nb,),
        in_specs=[
            pl.BlockSpec((1, _RB, D), lambda i: (0, i, 0)),
            pl.BlockSpec((1, _RB, D), lambda i: (1, i, 0)),
            pl.BlockSpec((_RB, D), lambda i: (i, 0)),
            pl.BlockSpec((_RB, 1), lambda i: (i, 0)),
            pl.BlockSpec((_RB, 1), lambda i: (i, 0)),
            pl.BlockSpec((_RB, 1), lambda i: (i, 0)),
            pl.BlockSpec((8, 128), lambda i: (0, 0)),
            pl.BlockSpec((8, 128), lambda i: (0, 0)),
        ],
        out_specs=[
            pl.BlockSpec((_RB, D), lambda i: (i, 0)),
            pl.BlockSpec((_RB, 1), lambda i: (i, 0)),
        ],
        out_shape=[jax.ShapeDtypeStruct((Np, D), jnp.float32),
                   jax.ShapeDtypeStruct((Np, 1), jnp.float32)],
    )(acc_p, acc_p, xs, dinv_c, sd0c, sd1c, b_pad, sparams)


def _tc_topk(s_row, s_col, n_real, k, kp_next, L):
    NEG = -3e38

    def body(sr_ref, sc_ref, ni_ref):
        g = pl.program_id(0)
        srow = sr_ref[0, 0, :]
        scol = sc_ref[0, :, :]
        li = lax.broadcasted_iota(jnp.int32, (L,), 0)
        real_r = li < n_real
        srow = jnp.where(real_r, srow, NEG)
        cj = lax.broadcasted_iota(jnp.int32, (L, 1), 0)
        real_c = cj < n_real
        scol = jnp.where(real_c, scol, NEG)
        jj = lax.broadcasted_iota(jnp.int32, (L, L), 0)
        ii = lax.broadcasted_iota(jnp.int32, (L, L), 1)
        beats = (scol > srow) | ((scol == srow) & (jj < ii))
        cnt = jnp.sum(beats.astype(jnp.float32), axis=0).astype(jnp.int32)
        sel = real_r & (cnt < k)
        ni_ref[0, 0, :] = jnp.where(sel, g * kp_next + cnt, -1)

    return pl.pallas_call(
        body,
        grid=(G,),
        in_specs=[
            pl.BlockSpec((1, 1, L), lambda g: (g, 0, 0)),
            pl.BlockSpec((1, L, 1), lambda g: (g, 0, 0)),
        ],
        out_specs=pl.BlockSpec((1, 1, L), lambda g: (g, 0, 0)),
        out_shape=jax.ShapeDtypeStruct((G, 1, L), jnp.int32),
    )(s_row, s_col)


def _tc_final(x1, x2, x3, lin1_W, lin1_b, lin2_W, lin2_b, lin3_W, lin3_b):
    def body(x1_ref, x2_ref, x3_ref, w1_ref, b1_ref, w2_ref, b2_ref,
             w3_ref, b3_ref, out_ref):
        def readout(ref, k, kp):
            mx, mn = [], []
            for g in range(G):
                xg = ref[g * kp:g * kp + k, :]
                mx.append(jnp.max(xg, axis=0, keepdims=True))
                mn.append(jnp.sum(xg, axis=0, keepdims=True) / k)
            return jnp.concatenate(
                [jnp.concatenate(mx, axis=0), jnp.concatenate(mn, axis=0)],
                axis=1)

        r = (readout(x1_ref, 625, 640) + readout(x2_ref, 313, 320)
             + readout(x3_ref, 157, 160))
        h = jnp.maximum(jnp.dot(r, w1_ref[...],
                                preferred_element_type=jnp.float32)
                        + b1_ref[0:1, :], 0.0)
        h = jnp.maximum(jnp.dot(h, w2_ref[...],
                                preferred_element_type=jnp.float32)
                        + b2_ref[0:1, :64], 0.0)
        z = jnp.dot(h, w3_ref[...], preferred_element_type=jnp.float32) \
            + b3_ref[0:1, :10]
        m = jnp.max(z, axis=1, keepdims=True)
        e = jnp.exp(z - m)
        out_ref[...] = (z - m) - jnp.log(jnp.sum(e, axis=1, keepdims=True))

    return pl.pallas_call(
        body,
        out_shape=jax.ShapeDtypeStruct((G, 10), jnp.float32),
    )(x1, x2, x3, lin1_W, lin1_b, lin2_W, lin2_b, lin3_W, lin3_b)


def _blockdiag(W):
    H, di, do = W.shape
    M = jnp.zeros((H * di, H * do), jnp.float32)
    for i in range(H):
        M = M.at[i * di:(i + 1) * di, i * do:(i + 1) * do].set(W[i])
    return M


def _pad128(v):
    return jnp.pad(jnp.broadcast_to(v.reshape(1, -1), (8, v.shape[-1])),
                   ((0, 0), (0, 128 - v.shape[-1])))


def kernel(x, edge_index, batch,
           conv_W1, conv_b1, att_W1, att_b1, score_W1, score_b1,
           conv_W2, conv_b2, att_W2, att_b2, score_W2, score_b2,
           conv_W3, conv_b3, att_W3, att_b3, score_W3, score_b3,
           lin1_W, lin1_b, lin2_W, lin2_b, lin3_W, lin3_b):
    src = jnp.pad(edge_index[0].astype(jnp.int32), (0, EP - E0),
                  constant_values=-1)
    dst = jnp.pad(edge_index[1].astype(jnp.int32), (0, EP - E0),
                  constant_values=-1)
    pck = jnp.concatenate([src, dst])

    convs = ((conv_W1, conv_b1, score_W1, score_b1),
             (conv_W2, conv_b2, score_W2, score_b2),
             (conv_W3, conv_b3, score_W3, score_b3))

    xcur = jnp.pad(x, ((0, STAGES[0]["Np"] - N0), (0, 0)))
    ni_prev = None
    readouts = []

    for si, cfg in enumerate(STAGES):
        Np, stride, n_real = cfg["Np"], cfg["stride"], cfg["n_real"]
        k, kp_next, L = cfg["k"], cfg["kp_next"], cfg["L"]
        cW, cb, sW, sb = convs[si]

        if si == 0:
            pass_a = _make_pass_a(Np, False, 0)
            (deg_p,) = pass_a(pck)
            cnts = None
            cnts_prev = None
        else:
            pass_a = _make_pass_a(Np, True, ni_prev.shape[0])
            if cnts_prev is None:
                cnts_prev = jnp.full((NW * 16,), EW, jnp.int32)
            pck, cnts, x_new_prev, deg_p = pass_a(pck, ni_prev, cnts_prev,
                                                  hs_prev)
            readouts.append(x_new_prev)
            xcur = x_new_prev[:Np]

        acc_rows = Np + TRASH
        d0c = deg_p[:Np].reshape(Np, 1)
        d1c = deg_p[acc_rows:acc_rows + Np].reshape(Np, 1)
        Wbd = _blockdiag(cW)
        xs, dinv_c = _tc_prep(xcur, Wbd, d0c, d1c, Np)

        pass_b = _make_pass_b(Np, dynamic=si > 0)
        if si == 0:
            acc_p, sdeg_p = pass_b(pck, xs, dinv_c.reshape(Np))
        else:
            acc_p, sdeg_p = pass_b(pck, cnts, xs, dinv_c.reshape(Np))

        b_cat = jnp.concatenate([cb[i] for i in range(4)])
        sparams = _pad128(jnp.stack([sW[0, 0], sb[0]]))
        sd0c = sdeg_p[:Np].reshape(Np, 1)
        sd1c = sdeg_p[acc_rows:acc_rows + Np].reshape(Np, 1)
        hs, score_c = _tc_post(acc_p, xs, dinv_c, sd0c, sd1c, _pad128(b_cat),
                               sparams, Np)

        sflat = score_c.reshape(Np)
        if si == 0:
            sreal = sflat[:N0].reshape(G, stride)
            sreal = jnp.pad(sreal, ((0, 0), (0, L - stride)),
                            constant_values=-3e38)
        else:
            sreal = sflat.reshape(G, L)
        ni3 = _tc_topk(sreal.reshape(G, 1, L), sreal.reshape(G, L, 1),
                       n_real, k, kp_next, L)

        if si == 0:
            ni_flat = jnp.pad(ni3[:, 0, :stride].reshape(-1),
                              (0, Np - N0), constant_values=-1)
        else:
            ni_flat = ni3[:, 0, :].reshape(-1)

        hs_prev = hs
        ni_prev = ni_flat
        cnts_prev = cnts

    pass_c = _make_pass_c(STAGES[2]["Np"], G * STAGES[2]["kp_next"])
    (x_new3,) = pass_c(hs_prev, ni_prev)
    readouts.append(x_new3)

    out = _tc_final(readouts[0], readouts[1], readouts[2],
                    lin1_W, _pad128(lin1_b), lin2_W, _pad128(lin2_b),
                    lin3_W, _pad128(lin3_b))
    return out
